# Initial kernel scaffold; baseline (speedup 1.0000x reference)
import functools

import jax
import jax.numpy as jnp
from jax import lax
from jax.experimental import pallas as pl
from jax.experimental.pallas import tpu as pltpu
from jax.experimental.pallas import tpu_sc as plsc

N = 10000
D_IN = 256
H1 = 8
C1 = 8
OUT = 64
E = 160000

NC = 2
NS = 16
CH = 128
CHUNKS = 40
EP = NC * NS * CHUNKS * CH
NP = 10016
STRIPE = NP // NS

BLK = 1000
GRID = N // BLK

_f32 = jnp.float32


def _sel_mat(rows, cols):
    if rows < cols:
        return (lax.broadcasted_iota(jnp.int32, (rows, cols), 0)
                == lax.broadcasted_iota(jnp.int32, (rows, cols), 1)
                // (cols // rows)).astype(_f32)
    else:
        return (lax.broadcasted_iota(jnp.int32, (rows, cols), 0)
                // (rows // cols)
                == lax.broadcasted_iota(jnp.int32, (rows, cols), 1)
                ).astype(_f32)


def _tc1_body(x_ref, w1_ref, ats_ref, atd_ref, h_ref, as_ref, ad_ref,
              eas_ref):
    h = jnp.dot(x_ref[...], w1_ref[...], preferred_element_type=_f32)
    h_ref[...] = h
    sel = _sel_mat(64, 8)
    a_s = jnp.dot(h * ats_ref[...], sel, preferred_element_type=_f32)
    a_d = jnp.dot(h * atd_ref[...], sel, preferred_element_type=_f32)
    as_ref[...] = a_s
    ad_ref[...] = a_d
    al = a_s + a_d
    eas_ref[...] = jnp.exp(jnp.where(al >= 0, al, 0.2 * al))


def _tc1(x, w1, ats, atd):
    return pl.pallas_call(
        _tc1_body,
        grid=(GRID,),
        in_specs=[
            pl.BlockSpec((BLK, D_IN), lambda i: (i, 0)),
            pl.BlockSpec((D_IN, 64), lambda i: (0, 0)),
            pl.BlockSpec((1, 64), lambda i: (0, 0)),
            pl.BlockSpec((1, 64), lambda i: (0, 0)),
        ],
        out_specs=[
            pl.BlockSpec((BLK, 64), lambda i: (i, 0)),
            pl.BlockSpec((BLK, 8), lambda i: (i, 0)),
            pl.BlockSpec((BLK, 8), lambda i: (i, 0)),
            pl.BlockSpec((BLK, 8), lambda i: (i, 0)),
        ],
        out_shape=[
            jax.ShapeDtypeStruct((N, 64), _f32),
            jax.ShapeDtypeStruct((N, 8), _f32),
            jax.ShapeDtypeStruct((N, 8), _f32),
            jax.ShapeDtypeStruct((N, 8), _f32),
        ],
    )(x, w1, ats, atd)


def _tc2_body(m0_ref, m1_ref, s0_ref, s1_ref, eas_ref, h1_ref, w2_ref,
              at2s_ref, at2d_ref, b1_ref, h2_ref, as2_ref, ad2_ref):
    sel = _sel_mat(8, 64)
    eas = eas_ref[...]
    s64 = jnp.dot(s0_ref[...] + s1_ref[...] + eas, sel,
                  preferred_element_type=_f32)
    num = (m0_ref[...] + m1_ref[...]
           + h1_ref[...] * jnp.dot(eas, sel, preferred_element_type=_f32))
    o1 = num / (s64 + 1e-16) + b1_ref[...]
    h1p = jnp.where(o1 > 0, o1, jnp.exp(jnp.minimum(o1, 0.0)) - 1.0)
    h2 = jnp.dot(h1p, w2_ref[...], preferred_element_type=_f32)
    h2_ref[...] = h2
    as2_ref[...] = jnp.sum(h2 * at2s_ref[...], axis=1, keepdims=True)
    ad2_ref[...] = jnp.sum(h2 * at2d_ref[...], axis=1, keepdims=True)


def _tc2(m0, m1, s0, s1, eas, h1, w2, at2s, at2d, b1):
    return pl.pallas_call(
        _tc2_body,
        grid=(GRID,),
        in_specs=[
            pl.BlockSpec((BLK, 64), lambda i: (i, 0)),
            pl.BlockSpec((BLK, 64), lambda i: (i, 0)),
            pl.BlockSpec((BLK, 8), lambda i: (i, 0)),
            pl.BlockSpec((BLK, 8), lambda i: (i, 0)),
            pl.BlockSpec((BLK, 8), lambda i: (i, 0)),
            pl.BlockSpec((BLK, 64), lambda i: (i, 0)),
            pl.BlockSpec((64, 64), lambda i: (0, 0)),
            pl.BlockSpec((1, 64), lambda i: (0, 0)),
            pl.BlockSpec((1, 64), lambda i: (0, 0)),
            pl.BlockSpec((1, 64), lambda i: (0, 0)),
        ],
        out_specs=[
            pl.BlockSpec((BLK, 64), lambda i: (i, 0)),
            pl.BlockSpec((BLK, 1), lambda i: (i, 0)),
            pl.BlockSpec((BLK, 1), lambda i: (i, 0)),
        ],
        out_shape=[
            jax.ShapeDtypeStruct((N, 64), _f32),
            jax.ShapeDtypeStruct((N, 1), _f32),
            jax.ShapeDtypeStruct((N, 1), _f32),
        ],
    )(m0, m1, s0, s1, eas, h1, w2, at2s, at2d, b1)


def _tc3_body(m0_ref, m1_ref, s0_ref, s1_ref, as2_ref, ad2_ref, h2_ref,
              b2_ref, out_ref):
    al = as2_ref[...] + ad2_ref[...]
    eas2 = jnp.exp(jnp.where(al >= 0, al, 0.2 * al))
    s = s0_ref[...] + s1_ref[...] + eas2
    num = m0_ref[...] + m1_ref[...] + h2_ref[...] * eas2
    o = num / (s + 1e-16) + b2_ref[...]
    z = o - jnp.max(o, axis=1, keepdims=True)
    out_ref[...] = z - jnp.log(jnp.sum(jnp.exp(z), axis=1, keepdims=True))


def _tc3(m0, m1, s0, s1, as2, ad2, h2, b2):
    return pl.pallas_call(
        _tc3_body,
        grid=(GRID,),
        in_specs=[
            pl.BlockSpec((BLK, 64), lambda i: (i, 0)),
            pl.BlockSpec((BLK, 64), lambda i: (i, 0)),
            pl.BlockSpec((BLK, 1), lambda i: (i, 0)),
            pl.BlockSpec((BLK, 1), lambda i: (i, 0)),
            pl.BlockSpec((BLK, 1), lambda i: (i, 0)),
            pl.BlockSpec((BLK, 1), lambda i: (i, 0)),
            pl.BlockSpec((BLK, 64), lambda i: (i, 0)),
            pl.BlockSpec((1, 64), lambda i: (0, 0)),
        ],
        out_specs=pl.BlockSpec((BLK, 64), lambda i: (i, 0)),
        out_shape=jax.ShapeDtypeStruct((N, 64), _f32),
    )(m0, m1, s0, s1, as2, ad2, h2, b2)


@functools.partial(
    pl.kernel,
    out_type=jax.ShapeDtypeStruct((NC, NP, 72), _f32),
    mesh=plsc.VectorSubcoreMesh(core_axis_name="c", subcore_axis_name="s"),
    scratch_types=[
        pltpu.VMEM_SHARED((NP, 64), _f32),
        pltpu.VMEM_SHARED((NP, 8), _f32),
        pltpu.VMEM_SHARED((NP, 8), _f32),
        pltpu.VMEM_SHARED((NP, 72), _f32),
        pltpu.VMEM((CH,), jnp.int32),
        pltpu.VMEM((CH,), jnp.int32),
        pltpu.VMEM((CH, 8), _f32),
        pltpu.VMEM((CH, 8), _f32),
        pltpu.VMEM((CH, 64), _f32),
        pltpu.VMEM((CH, 72), _f32),
        pltpu.SemaphoreType.DMA,
        pltpu.SemaphoreType.DMA,
        pltpu.SemaphoreType.DMA,
    ],
)
def _sc_layer1(h_hbm, as_hbm, ad_hbm, src_hbm, dst_hbm, z_hbm, out_hbm,
               h_sh, as_sh, ad_sh, acc_sh, src_v, dst_v, asr_v, adr_v,
               hr_v, msg_v, sem0, sem1, sem2):
    cid = lax.axis_index("c")
    sid = lax.axis_index("s")
    r0 = sid * STRIPE
    pltpu.sync_copy(h_hbm.at[pl.ds(r0, STRIPE)], h_sh.at[pl.ds(r0, STRIPE)])
    pltpu.sync_copy(as_hbm.at[pl.ds(r0, STRIPE)], as_sh.at[pl.ds(r0, STRIPE)])
    pltpu.sync_copy(ad_hbm.at[pl.ds(r0, STRIPE)], ad_sh.at[pl.ds(r0, STRIPE)])
    pltpu.sync_copy(z_hbm, acc_sh.at[pl.ds(r0, STRIPE)])
    plsc.subcore_barrier()

    i16 = lax.broadcasted_iota(jnp.int32, (16,), 0)
    p8 = i16 // 8
    e8 = i16 - 8 * p8
    base0 = (cid * NS + sid) * (CHUNKS * CH)

    def chunk_body(j, carry):
        base = base0 + j * CH
        pltpu.sync_copy(src_hbm.at[pl.ds(base, CH)], src_v)
        pltpu.sync_copy(dst_hbm.at[pl.ds(base, CH)], dst_v)
        cp_a = pltpu.async_copy(as_sh.at[src_v], asr_v, sem0)
        cp_b = pltpu.async_copy(ad_sh.at[dst_v], adr_v, sem1)
        cp_h = pltpu.async_copy(h_sh.at[src_v], hr_v, sem2)
        cp_a.wait()
        cp_b.wait()

        def ea_body(v, c):
            rows = 2 * v + p8
            a = (plsc.load_gather(asr_v, [rows, e8])
                 + plsc.load_gather(adr_v, [rows, e8]))
            a = jnp.where(a >= 0, a, 0.2 * a)
            plsc.store_scatter(msg_v, [rows, 64 + e8], jnp.exp(a))
            return c
        lax.fori_loop(0, CH // 2, ea_body, 0)

        cp_h.wait()

        def mul_body(e, c):
            erow = jnp.full((16,), 0, jnp.int32) + e
            for k in range(4):
                mult = plsc.load_gather(msg_v, [erow, 64 + 2 * k + p8])
                hv = plsc.load_gather(hr_v, [erow, 16 * k + i16])
                plsc.store_scatter(msg_v, [erow, 16 * k + i16], hv * mult)
            return c
        lax.fori_loop(0, CH, mul_body, 0)

        pltpu.sync_copy(msg_v, acc_sh.at[dst_v], add=True)
        return carry

    lax.fori_loop(0, CHUNKS, chunk_body, 0)
    plsc.subcore_barrier()
    pltpu.sync_copy(acc_sh.at[pl.ds(r0, STRIPE)],
                    out_hbm.at[cid, pl.ds(r0, STRIPE)])


@functools.partial(
    pl.kernel,
    out_type=jax.ShapeDtypeStruct((NC, NP, 65), _f32),
    mesh=plsc.VectorSubcoreMesh(core_axis_name="c", subcore_axis_name="s"),
    scratch_types=[
        pltpu.VMEM_SHARED((NP, 64), _f32),
        pltpu.VMEM_SHARED((NP, 65), _f32),
        pltpu.VMEM((NP,), _f32),
        pltpu.VMEM((NP,), _f32),
        pltpu.VMEM((CH,), jnp.int32),
        pltpu.VMEM((CH,), jnp.int32),
        pltpu.VMEM((CH,), _f32),
        pltpu.VMEM((CH, 64), _f32),
        pltpu.VMEM((CH, 65), _f32),
        pltpu.SemaphoreType.DMA,
    ],
)
def _sc_layer2(h_hbm, as_hbm, ad_hbm, src_hbm, dst_hbm, z_hbm, out_hbm,
               h_sh, acc_sh, as_t, ad_t, src_v, dst_v, ea_v, hr_v, msg_v,
               sem0):
    cid = lax.axis_index("c")
    sid = lax.axis_index("s")
    r0 = sid * STRIPE
    pltpu.sync_copy(h_hbm.at[pl.ds(r0, STRIPE)], h_sh.at[pl.ds(r0, STRIPE)])
    pltpu.sync_copy(as_hbm, as_t)
    pltpu.sync_copy(ad_hbm, ad_t)
    pltpu.sync_copy(z_hbm, acc_sh.at[pl.ds(r0, STRIPE)])
    plsc.subcore_barrier()

    i16 = lax.broadcasted_iota(jnp.int32, (16,), 0)
    c64 = jnp.full((16,), 0, jnp.int32) + 64
    base0 = (cid * NS + sid) * (CHUNKS * CH)

    def chunk_body(j, carry):
        base = base0 + j * CH
        pltpu.sync_copy(src_hbm.at[pl.ds(base, CH)], src_v)
        pltpu.sync_copy(dst_hbm.at[pl.ds(base, CH)], dst_v)
        cp_h = pltpu.async_copy(h_sh.at[src_v], hr_v, sem0)

        def ea_body(v, c):
            sv = src_v[pl.ds(v * 16, 16)]
            dv = dst_v[pl.ds(v * 16, 16)]
            a = plsc.load_gather(as_t, [sv]) + plsc.load_gather(ad_t, [dv])
            a = jnp.where(a >= 0, a, 0.2 * a)
            ea = jnp.exp(a)
            ea_v[pl.ds(v * 16, 16)] = ea
            plsc.store_scatter(msg_v, [v * 16 + i16, c64], ea)
            return c
        lax.fori_loop(0, CH // 16, ea_body, 0)

        cp_h.wait()

        def mul_body(e, c):
            erow = jnp.full((16,), 0, jnp.int32) + e
            mult = plsc.load_gather(ea_v, [erow])
            for k in range(4):
                hv = plsc.load_gather(hr_v, [erow, 16 * k + i16])
                plsc.store_scatter(msg_v, [erow, 16 * k + i16], hv * mult)
            return c
        lax.fori_loop(0, CH, mul_body, 0)

        pltpu.sync_copy(msg_v, acc_sh.at[dst_v], add=True)
        return carry

    lax.fori_loop(0, CHUNKS, chunk_body, 0)
    plsc.subcore_barrier()
    pltpu.sync_copy(acc_sh.at[pl.ds(r0, STRIPE)],
                    out_hbm.at[cid, pl.ds(r0, STRIPE)])


def kernel(x, edge_index, W1, att_src1, att_dst1, b1, W2, att_src2,
           att_dst2, b2):
    src = edge_index[0]
    dst = edge_index[1]
    pad = jnp.full((EP - E,), N, jnp.int32)
    srcp = jnp.concatenate([src, pad])
    dstp = jnp.concatenate([dst, pad])

    h1, as1, ad1, eas1 = _tc1(x, W1, att_src1.reshape(1, 64),
                              att_dst1.reshape(1, 64))

    rpad = ((0, NP - N), (0, 0))
    o1 = _sc_layer1(jnp.pad(h1, rpad), jnp.pad(as1, rpad),
                    jnp.pad(ad1, rpad), srcp, dstp,
                    jnp.zeros((STRIPE, 72), _f32))

    h2, as2, ad2 = _tc2(o1[0, :N, :64], o1[1, :N, :64],
                        o1[0, :N, 64:], o1[1, :N, 64:],
                        eas1, h1, W2, att_src2.reshape(1, 64),
                        att_dst2.reshape(1, 64), b1.reshape(1, 64))

    o2 = _sc_layer2(jnp.pad(h2, rpad), jnp.pad(as2.reshape(-1), (0, NP - N)),
                    jnp.pad(ad2.reshape(-1), (0, NP - N)), srcp, dstp,
                    jnp.zeros((STRIPE, 65), _f32))

    return _tc3(o2[0, :N, :64], o2[1, :N, :64],
                o2[0, :N, 64:], o2[1, :N, 64:],
                as2, ad2, h2, b2.reshape(1, 64))


# speedup vs baseline: 32.4749x; 32.4749x over previous
import functools

import jax
import jax.numpy as jnp
from jax import lax
from jax.experimental import pallas as pl
from jax.experimental.pallas import tpu as pltpu
from jax.experimental.pallas import tpu_sc as plsc

N = 10000
D_IN = 256
OUT = 64
E = 160000

NC = 2
NS = 16
CH = 128
CHUNKS = 40
EP = NC * NS * CHUNKS * CH
NP = 10112
STRIPE = NP // NS

BLK = 1000
GRID = N // BLK

_f32 = jnp.float32

_SC_PARAMS = pltpu.CompilerParams(
    needs_layout_passes=False,
    use_tc_tiling_on_sc=False,
)


def _sel_mat(rows, cols):
    if rows < cols:
        return (lax.broadcasted_iota(jnp.int32, (rows, cols), 0)
                == lax.broadcasted_iota(jnp.int32, (rows, cols), 1)
                // (cols // rows)).astype(_f32)
    else:
        return (lax.broadcasted_iota(jnp.int32, (rows, cols), 0)
                // (rows // cols)
                == lax.broadcasted_iota(jnp.int32, (rows, cols), 1)
                ).astype(_f32)


def _tc1_body(x_ref, w1_ref, ats_ref, atd_ref, h_ref, as_ref, ad_ref,
              eas_ref):
    h = jnp.dot(x_ref[...], w1_ref[...], preferred_element_type=_f32)
    h_ref[...] = h
    sel = _sel_mat(64, 8)
    a_s = jnp.dot(h * ats_ref[...], sel, preferred_element_type=_f32)
    a_d = jnp.dot(h * atd_ref[...], sel, preferred_element_type=_f32)
    as_ref[...] = a_s
    ad_ref[...] = a_d
    al = a_s + a_d
    eas_ref[...] = jnp.exp(jnp.where(al >= 0, al, 0.2 * al))


def _tc1(x, w1, ats, atd):
    return pl.pallas_call(
        _tc1_body,
        grid=(GRID,),
        in_specs=[
            pl.BlockSpec((BLK, D_IN), lambda i: (i, 0)),
            pl.BlockSpec((D_IN, 64), lambda i: (0, 0)),
            pl.BlockSpec((1, 64), lambda i: (0, 0)),
            pl.BlockSpec((1, 64), lambda i: (0, 0)),
        ],
        out_specs=[
            pl.BlockSpec((BLK, 64), lambda i: (i, 0)),
            pl.BlockSpec((BLK, 8), lambda i: (i, 0)),
            pl.BlockSpec((BLK, 8), lambda i: (i, 0)),
            pl.BlockSpec((BLK, 8), lambda i: (i, 0)),
        ],
        out_shape=[
            jax.ShapeDtypeStruct((N, 64), _f32),
            jax.ShapeDtypeStruct((N, 8), _f32),
            jax.ShapeDtypeStruct((N, 8), _f32),
            jax.ShapeDtypeStruct((N, 8), _f32),
        ],
    )(x, w1, ats, atd)


def _tc2_body(m0_ref, m1_ref, s0_ref, s1_ref, eas_ref, h1_ref, w2_ref,
              at2s_ref, at2d_ref, b1_ref, h2_ref, as2_ref, ad2_ref):
    sel = _sel_mat(8, 64)
    eas = eas_ref[...]
    s64 = jnp.dot(s0_ref[...] + s1_ref[...] + eas, sel,
                  preferred_element_type=_f32)
    num = (m0_ref[...] + m1_ref[...]
           + h1_ref[...] * jnp.dot(eas, sel, preferred_element_type=_f32))
    o1 = num / (s64 + 1e-16) + b1_ref[...]
    h1p = jnp.where(o1 > 0, o1, jnp.exp(jnp.minimum(o1, 0.0)) - 1.0)
    h2 = jnp.dot(h1p, w2_ref[...], preferred_element_type=_f32)
    h2_ref[...] = h2
    as2_ref[...] = jnp.sum(h2 * at2s_ref[...], axis=1, keepdims=True)
    ad2_ref[...] = jnp.sum(h2 * at2d_ref[...], axis=1, keepdims=True)


def _tc2(m0, m1, s0, s1, eas, h1, w2, at2s, at2d, b1):
    return pl.pallas_call(
        _tc2_body,
        grid=(GRID,),
        in_specs=[
            pl.BlockSpec((BLK, 64), lambda i: (i, 0)),
            pl.BlockSpec((BLK, 64), lambda i: (i, 0)),
            pl.BlockSpec((BLK, 8), lambda i: (i, 0)),
            pl.BlockSpec((BLK, 8), lambda i: (i, 0)),
            pl.BlockSpec((BLK, 8), lambda i: (i, 0)),
            pl.BlockSpec((BLK, 64), lambda i: (i, 0)),
            pl.BlockSpec((64, 64), lambda i: (0, 0)),
            pl.BlockSpec((1, 64), lambda i: (0, 0)),
            pl.BlockSpec((1, 64), lambda i: (0, 0)),
            pl.BlockSpec((1, 64), lambda i: (0, 0)),
        ],
        out_specs=[
            pl.BlockSpec((BLK, 64), lambda i: (i, 0)),
            pl.BlockSpec((BLK, 1), lambda i: (i, 0)),
            pl.BlockSpec((BLK, 1), lambda i: (i, 0)),
        ],
        out_shape=[
            jax.ShapeDtypeStruct((N, 64), _f32),
            jax.ShapeDtypeStruct((N, 1), _f32),
            jax.ShapeDtypeStruct((N, 1), _f32),
        ],
    )(m0, m1, s0, s1, eas, h1, w2, at2s, at2d, b1)


def _tc3_body(m0_ref, m1_ref, s0_ref, s1_ref, as2_ref, ad2_ref, h2_ref,
              b2_ref, out_ref):
    al = as2_ref[...] + ad2_ref[...]
    eas2 = jnp.exp(jnp.where(al >= 0, al, 0.2 * al))
    s = s0_ref[...] + s1_ref[...] + eas2
    num = m0_ref[...] + m1_ref[...] + h2_ref[...] * eas2
    o = num / (s + 1e-16) + b2_ref[...]
    z = o - jnp.max(o, axis=1, keepdims=True)
    out_ref[...] = z - jnp.log(jnp.sum(jnp.exp(z), axis=1, keepdims=True))


def _tc3(m0, m1, s0, s1, as2, ad2, h2, b2):
    return pl.pallas_call(
        _tc3_body,
        grid=(GRID,),
        in_specs=[
            pl.BlockSpec((BLK, 64), lambda i: (i, 0)),
            pl.BlockSpec((BLK, 64), lambda i: (i, 0)),
            pl.BlockSpec((BLK, 1), lambda i: (i, 0)),
            pl.BlockSpec((BLK, 1), lambda i: (i, 0)),
            pl.BlockSpec((BLK, 1), lambda i: (i, 0)),
            pl.BlockSpec((BLK, 1), lambda i: (i, 0)),
            pl.BlockSpec((BLK, 64), lambda i: (i, 0)),
            pl.BlockSpec((1, 64), lambda i: (0, 0)),
        ],
        out_specs=pl.BlockSpec((BLK, 64), lambda i: (i, 0)),
        out_shape=jax.ShapeDtypeStruct((N, 64), _f32),
    )(m0, m1, s0, s1, as2, ad2, h2, b2)


def _sc_edge_pass(nea):
    w = 72

    @functools.partial(
        pl.kernel,
        out_type=jax.ShapeDtypeStruct((NC, NP, w), _f32),
        mesh=plsc.VectorSubcoreMesh(core_axis_name="c", subcore_axis_name="s"),
        compiler_params=_SC_PARAMS,
        scratch_types=[
            pltpu.VMEM_SHARED((NP, w), _f32),
            pltpu.VMEM((CH,), jnp.int32),
            pltpu.VMEM((CH,), jnp.int32),
            pltpu.VMEM((CH, 8), _f32),
            pltpu.VMEM((CH, 8), _f32),
            pltpu.VMEM((CH, 64), _f32),
            pltpu.VMEM((CH, w), _f32),
            pltpu.SemaphoreType.DMA,
            pltpu.SemaphoreType.DMA,
            pltpu.SemaphoreType.DMA,
        ],
    )
    def edge_pass(h_hbm, as_hbm, ad_hbm, src_hbm, dst_hbm, z_hbm, out_hbm,
                  acc_sh, src_v, dst_v, asr_v, adr_v, hr_v, msg_v,
                  sem0, sem1, sem2):
        cid = lax.axis_index("c")
        sid = lax.axis_index("s")
        r0 = sid * STRIPE
        pltpu.sync_copy(z_hbm, acc_sh.at[pl.ds(r0, STRIPE)])
        plsc.subcore_barrier()

        i16 = lax.broadcasted_iota(jnp.int32, (16,), 0)
        p8 = i16 // 8
        e8 = i16 - 8 * p8
        base0 = (cid * NS + sid) * (CHUNKS * CH)

        if nea != 8:
            def z_body(e, c):
                plsc.store_scatter(msg_v, [i16 * 0 + e, 56 + i16],
                                   jnp.zeros((16,), _f32))
                return c
            lax.fori_loop(0, CH, z_body, 0)

        def chunk_body(j, carry):
            base = base0 + j * CH
            pltpu.sync_copy(src_hbm.at[pl.ds(base, CH)], src_v)
            pltpu.sync_copy(dst_hbm.at[pl.ds(base, CH)], dst_v)
            cp_a = pltpu.async_copy(as_hbm.at[src_v], asr_v, sem0)
            cp_b = pltpu.async_copy(ad_hbm.at[dst_v], adr_v, sem1)
            cp_h = pltpu.async_copy(h_hbm.at[src_v], hr_v, sem2)
            cp_a.wait()
            cp_b.wait()

            if nea == 8:
                def ea_body(v, c):
                    rows = 2 * v + p8
                    a = (plsc.load_gather(asr_v, [rows, e8])
                         + plsc.load_gather(adr_v, [rows, e8]))
                    a = jnp.where(a >= 0, a, 0.2 * a)
                    plsc.store_scatter(msg_v, [rows, 64 + e8], jnp.exp(a))
                    return c
                lax.fori_loop(0, CH // 2, ea_body, 0)
            else:
                z16 = i16 * 0
                def ea_body(v, c):
                    rows = 16 * v + i16
                    a = (plsc.load_gather(asr_v, [rows, z16])
                         + plsc.load_gather(adr_v, [rows, z16]))
                    a = jnp.where(a >= 0, a, 0.2 * a)
                    plsc.store_scatter(msg_v, [rows, z16 + 64], jnp.exp(a))
                    return c
                lax.fori_loop(0, CH // 16, ea_body, 0)

            cp_h.wait()

            if nea == 8:
                def mul_body(e, c):
                    erow = i16 * 0 + e
                    for k in range(4):
                        mult = plsc.load_gather(msg_v, [erow, 64 + 2 * k + p8])
                        hv = plsc.load_gather(hr_v, [erow, 16 * k + i16])
                        plsc.store_scatter(msg_v, [erow, 16 * k + i16],
                                           hv * mult)
                    return c
                lax.fori_loop(0, CH, mul_body, 0)
            else:
                z16 = i16 * 0
                def mul_body(e, c):
                    erow = i16 * 0 + e
                    mult = plsc.load_gather(msg_v, [erow, z16 + 64])
                    for k in range(4):
                        hv = plsc.load_gather(hr_v, [erow, 16 * k + i16])
                        plsc.store_scatter(msg_v, [erow, 16 * k + i16],
                                           hv * mult)
                    return c
                lax.fori_loop(0, CH, mul_body, 0)

            pltpu.sync_copy(msg_v, acc_sh.at[dst_v], add=True)
            return carry

        lax.fori_loop(0, CHUNKS, chunk_body, 0)
        plsc.subcore_barrier()
        pltpu.sync_copy(acc_sh.at[pl.ds(r0, STRIPE)],
                        out_hbm.at[cid, pl.ds(r0, STRIPE)])

    return edge_pass


_sc_layer1 = _sc_edge_pass(8)
_sc_layer2 = _sc_edge_pass(1)


def kernel(x, edge_index, W1, att_src1, att_dst1, b1, W2, att_src2,
           att_dst2, b2):
    src = edge_index[0]
    dst = edge_index[1]
    pad = jnp.full((EP - E,), N, jnp.int32)
    srcp = jnp.concatenate([src, pad])
    dstp = jnp.concatenate([dst, pad])

    h1, as1, ad1, eas1 = _tc1(x, W1, att_src1.reshape(1, 64),
                              att_dst1.reshape(1, 64))

    rpad = ((0, NP - N), (0, 0))
    o1 = _sc_layer1(jnp.pad(h1, rpad), jnp.pad(as1, rpad),
                    jnp.pad(ad1, rpad), srcp, dstp,
                    jnp.zeros((STRIPE, 72), _f32))

    h2, as2, ad2 = _tc2(o1[0, :N, :64], o1[1, :N, :64],
                        o1[0, :N, 64:], o1[1, :N, 64:],
                        eas1, h1, W2, att_src2.reshape(1, 64),
                        att_dst2.reshape(1, 64), b1.reshape(1, 64))

    cpad = ((0, NP - N), (0, 7))
    o2 = _sc_layer2(jnp.pad(h2, rpad), jnp.pad(as2, cpad),
                    jnp.pad(ad2, cpad), srcp, dstp,
                    jnp.zeros((STRIPE, 72), _f32))

    return _tc3(o2[0, :N, :64], o2[1, :N, :64],
                o2[0, :N, 64:65], o2[1, :N, 64:65],
                as2, ad2, h2, b2.reshape(1, 64))


# speedup vs baseline: 54.5371x; 1.6794x over previous
import functools

import jax
import jax.numpy as jnp
from jax import lax
from jax.experimental import pallas as pl
from jax.experimental.pallas import tpu as pltpu
from jax.experimental.pallas import tpu_sc as plsc

N = 10000
D_IN = 256
OUT = 64
E = 160000

NC = 2
NS = 16
CH = 128
CHUNKS = 40
EP = NC * NS * CHUNKS * CH
NP = 10112
STRIPE = NP // NS

BLK = 1000
GRID = N // BLK

_f32 = jnp.float32

_SC_PARAMS = pltpu.CompilerParams(
    needs_layout_passes=False,
    use_tc_tiling_on_sc=False,
)


def _sel_mat(rows, cols):
    if rows < cols:
        return (lax.broadcasted_iota(jnp.int32, (rows, cols), 0)
                == lax.broadcasted_iota(jnp.int32, (rows, cols), 1)
                // (cols // rows)).astype(_f32)
    else:
        return (lax.broadcasted_iota(jnp.int32, (rows, cols), 0)
                // (rows // cols)
                == lax.broadcasted_iota(jnp.int32, (rows, cols), 1)
                ).astype(_f32)


def _tc1_body(x_ref, w1_ref, ats_ref, atd_ref, h_ref, as_ref, ad_ref,
              eas_ref):
    h = jnp.dot(x_ref[...], w1_ref[...], preferred_element_type=_f32)
    h_ref[...] = h
    sel = _sel_mat(64, 8)
    a_s = jnp.dot(h * ats_ref[...], sel, preferred_element_type=_f32)
    a_d = jnp.dot(h * atd_ref[...], sel, preferred_element_type=_f32)
    as_ref[...] = a_s
    ad_ref[...] = a_d
    al = a_s + a_d
    eas_ref[...] = jnp.exp(jnp.where(al >= 0, al, 0.2 * al))


def _tc1(x, w1, ats, atd):
    return pl.pallas_call(
        _tc1_body,
        grid=(GRID,),
        in_specs=[
            pl.BlockSpec((BLK, D_IN), lambda i: (i, 0)),
            pl.BlockSpec((D_IN, 64), lambda i: (0, 0)),
            pl.BlockSpec((1, 64), lambda i: (0, 0)),
            pl.BlockSpec((1, 64), lambda i: (0, 0)),
        ],
        out_specs=[
            pl.BlockSpec((BLK, 64), lambda i: (i, 0)),
            pl.BlockSpec((BLK, 8), lambda i: (i, 0)),
            pl.BlockSpec((BLK, 8), lambda i: (i, 0)),
            pl.BlockSpec((BLK, 8), lambda i: (i, 0)),
        ],
        out_shape=[
            jax.ShapeDtypeStruct((N, 64), _f32),
            jax.ShapeDtypeStruct((N, 8), _f32),
            jax.ShapeDtypeStruct((N, 8), _f32),
            jax.ShapeDtypeStruct((N, 8), _f32),
        ],
    )(x, w1, ats, atd)


def _tc2_body(m0_ref, m1_ref, s0_ref, s1_ref, eas_ref, h1_ref, w2_ref,
              at2s_ref, at2d_ref, b1_ref, h2_ref, as2_ref, ad2_ref):
    sel = _sel_mat(8, 64)
    eas = eas_ref[...]
    s64 = jnp.dot(s0_ref[...] + s1_ref[...] + eas, sel,
                  preferred_element_type=_f32)
    num = (m0_ref[...] + m1_ref[...]
           + h1_ref[...] * jnp.dot(eas, sel, preferred_element_type=_f32))
    o1 = num / (s64 + 1e-16) + b1_ref[...]
    h1p = jnp.where(o1 > 0, o1, jnp.exp(jnp.minimum(o1, 0.0)) - 1.0)
    h2 = jnp.dot(h1p, w2_ref[...], preferred_element_type=_f32)
    h2_ref[...] = h2
    as2_ref[...] = jnp.sum(h2 * at2s_ref[...], axis=1, keepdims=True)
    ad2_ref[...] = jnp.sum(h2 * at2d_ref[...], axis=1, keepdims=True)


def _tc2(m0, m1, s0, s1, eas, h1, w2, at2s, at2d, b1):
    return pl.pallas_call(
        _tc2_body,
        grid=(GRID,),
        in_specs=[
            pl.BlockSpec((BLK, 64), lambda i: (i, 0)),
            pl.BlockSpec((BLK, 64), lambda i: (i, 0)),
            pl.BlockSpec((BLK, 8), lambda i: (i, 0)),
            pl.BlockSpec((BLK, 8), lambda i: (i, 0)),
            pl.BlockSpec((BLK, 8), lambda i: (i, 0)),
            pl.BlockSpec((BLK, 64), lambda i: (i, 0)),
            pl.BlockSpec((64, 64), lambda i: (0, 0)),
            pl.BlockSpec((1, 64), lambda i: (0, 0)),
            pl.BlockSpec((1, 64), lambda i: (0, 0)),
            pl.BlockSpec((1, 64), lambda i: (0, 0)),
        ],
        out_specs=[
            pl.BlockSpec((BLK, 64), lambda i: (i, 0)),
            pl.BlockSpec((BLK, 1), lambda i: (i, 0)),
            pl.BlockSpec((BLK, 1), lambda i: (i, 0)),
        ],
        out_shape=[
            jax.ShapeDtypeStruct((N, 64), _f32),
            jax.ShapeDtypeStruct((N, 1), _f32),
            jax.ShapeDtypeStruct((N, 1), _f32),
        ],
    )(m0, m1, s0, s1, eas, h1, w2, at2s, at2d, b1)


def _tc3_body(m0_ref, m1_ref, s0_ref, s1_ref, as2_ref, ad2_ref, h2_ref,
              b2_ref, out_ref):
    al = as2_ref[...] + ad2_ref[...]
    eas2 = jnp.exp(jnp.where(al >= 0, al, 0.2 * al))
    s = s0_ref[...] + s1_ref[...] + eas2
    num = m0_ref[...] + m1_ref[...] + h2_ref[...] * eas2
    o = num / (s + 1e-16) + b2_ref[...]
    z = o - jnp.max(o, axis=1, keepdims=True)
    out_ref[...] = z - jnp.log(jnp.sum(jnp.exp(z), axis=1, keepdims=True))


def _tc3(m0, m1, s0, s1, as2, ad2, h2, b2):
    return pl.pallas_call(
        _tc3_body,
        grid=(GRID,),
        in_specs=[
            pl.BlockSpec((BLK, 64), lambda i: (i, 0)),
            pl.BlockSpec((BLK, 64), lambda i: (i, 0)),
            pl.BlockSpec((BLK, 1), lambda i: (i, 0)),
            pl.BlockSpec((BLK, 1), lambda i: (i, 0)),
            pl.BlockSpec((BLK, 1), lambda i: (i, 0)),
            pl.BlockSpec((BLK, 1), lambda i: (i, 0)),
            pl.BlockSpec((BLK, 64), lambda i: (i, 0)),
            pl.BlockSpec((1, 64), lambda i: (0, 0)),
        ],
        out_specs=pl.BlockSpec((BLK, 64), lambda i: (i, 0)),
        out_shape=jax.ShapeDtypeStruct((N, 64), _f32),
    )(m0, m1, s0, s1, as2, ad2, h2, b2)


def _sc_edge_pass(nea):
    w = 72

    @functools.partial(
        pl.kernel,
        out_type=jax.ShapeDtypeStruct((NC, NP, w), _f32),
        mesh=plsc.VectorSubcoreMesh(core_axis_name="c", subcore_axis_name="s"),
        compiler_params=_SC_PARAMS,
        scratch_types=[
            pltpu.VMEM_SHARED((NP, w), _f32),
            pltpu.VMEM((2, CH), jnp.int32),
            pltpu.VMEM((2, CH), jnp.int32),
            pltpu.VMEM((2, CH, 8), _f32),
            pltpu.VMEM((2, CH, 8), _f32),
            pltpu.VMEM((2, CH, 64), _f32),
            pltpu.VMEM((CH, w), _f32),
            pltpu.SemaphoreType.DMA,
            pltpu.SemaphoreType.DMA,
            pltpu.SemaphoreType.DMA,
            pltpu.SemaphoreType.DMA,
            pltpu.SemaphoreType.DMA,
            pltpu.SemaphoreType.DMA,
        ],
    )
    def edge_pass(h_hbm, as_hbm, ad_hbm, src_hbm, dst_hbm, z_hbm, out_hbm,
                  acc_sh, src_v, dst_v, asr_v, adr_v, hr_v, msg_v,
                  *sems):
        cid = lax.axis_index("c")
        sid = lax.axis_index("s")
        r0 = sid * STRIPE
        pltpu.sync_copy(z_hbm, acc_sh.at[pl.ds(r0, STRIPE)])
        plsc.subcore_barrier()

        i16 = lax.broadcasted_iota(jnp.int32, (16,), 0)
        p8 = i16 // 8
        e8 = i16 - 8 * p8
        base0 = (cid * NS + sid) * (CHUNKS * CH)

        if nea != 8:
            def z_body(e):
                plsc.store_scatter(msg_v, [i16 * 0 + e, 56 + i16],
                                   jnp.zeros((16,), _f32))
            plsc.parallel_loop(0, CH, 1, unroll=4)(z_body)

        def prefetch(c, b):
            base = base0 + c * CH
            pltpu.sync_copy(src_hbm.at[pl.ds(base, CH)], src_v.at[b])
            pltpu.sync_copy(dst_hbm.at[pl.ds(base, CH)], dst_v.at[b])
            pltpu.async_copy(as_hbm.at[src_v.at[b]], asr_v.at[b], sems[3 * b])
            pltpu.async_copy(ad_hbm.at[dst_v.at[b]], adr_v.at[b],
                             sems[3 * b + 1])
            pltpu.async_copy(h_hbm.at[src_v.at[b]], hr_v.at[b],
                             sems[3 * b + 2])

        def compute(b):
            asr_b, adr_b, hr_b = asr_v.at[b], adr_v.at[b], hr_v.at[b]
            pltpu.make_async_copy(as_hbm.at[src_v.at[b]], asr_b,
                                  sems[3 * b]).wait()
            pltpu.make_async_copy(ad_hbm.at[dst_v.at[b]], adr_b,
                                  sems[3 * b + 1]).wait()

            if nea == 8:
                def ea_body(v):
                    rows = 2 * v + p8
                    a = (plsc.load_gather(asr_b, [rows, e8])
                         + plsc.load_gather(adr_b, [rows, e8]))
                    a = jnp.where(a >= 0, a, 0.2 * a)
                    plsc.store_scatter(msg_v, [rows, 64 + e8], jnp.exp(a))
                plsc.parallel_loop(0, CH // 2, 1, unroll=4)(ea_body)
            else:
                z16 = i16 * 0
                def ea_body(v):
                    rows = 16 * v + i16
                    a = (plsc.load_gather(asr_b, [rows, z16])
                         + plsc.load_gather(adr_b, [rows, z16]))
                    a = jnp.where(a >= 0, a, 0.2 * a)
                    plsc.store_scatter(msg_v, [rows, z16 + 64], jnp.exp(a))
                plsc.parallel_loop(0, CH // 16, 1, unroll=2)(ea_body)

            pltpu.make_async_copy(h_hbm.at[src_v.at[b]], hr_b,
                                  sems[3 * b + 2]).wait()

            if nea == 8:
                def mul_body(e):
                    erow = i16 * 0 + e
                    for k in range(4):
                        mult = plsc.load_gather(msg_v, [erow, 64 + 2 * k + p8])
                        hv = plsc.load_gather(hr_b, [erow, 16 * k + i16])
                        plsc.store_scatter(msg_v, [erow, 16 * k + i16],
                                           hv * mult)
                plsc.parallel_loop(0, CH, 1, unroll=4)(mul_body)
            else:
                z16 = i16 * 0
                def mul_body(e):
                    erow = i16 * 0 + e
                    mult = plsc.load_gather(msg_v, [erow, z16 + 64])
                    for k in range(4):
                        hv = plsc.load_gather(hr_b, [erow, 16 * k + i16])
                        plsc.store_scatter(msg_v, [erow, 16 * k + i16],
                                           hv * mult)
                plsc.parallel_loop(0, CH, 1, unroll=4)(mul_body)

            pltpu.sync_copy(msg_v, acc_sh.at[dst_v.at[b]], add=True)

        prefetch(0, 0)

        def chunk_body(i, carry):
            j = 2 * i
            prefetch(j + 1, 1)
            compute(0)

            @pl.when(j + 2 < CHUNKS)
            def _():
                prefetch(j + 2, 0)
            compute(1)
            return carry

        lax.fori_loop(0, CHUNKS // 2, chunk_body, 0)
        plsc.subcore_barrier()
        pltpu.sync_copy(acc_sh.at[pl.ds(r0, STRIPE)],
                        out_hbm.at[cid, pl.ds(r0, STRIPE)])

    return edge_pass


_sc_layer1 = _sc_edge_pass(8)
_sc_layer2 = _sc_edge_pass(1)


def kernel(x, edge_index, W1, att_src1, att_dst1, b1, W2, att_src2,
           att_dst2, b2):
    src = edge_index[0]
    dst = edge_index[1]
    pad = jnp.full((EP - E,), N, jnp.int32)
    srcp = jnp.concatenate([src, pad])
    dstp = jnp.concatenate([dst, pad])

    h1, as1, ad1, eas1 = _tc1(x, W1, att_src1.reshape(1, 64),
                              att_dst1.reshape(1, 64))

    rpad = ((0, NP - N), (0, 0))
    o1 = _sc_layer1(jnp.pad(h1, rpad), jnp.pad(as1, rpad),
                    jnp.pad(ad1, rpad), srcp, dstp,
                    jnp.zeros((STRIPE, 72), _f32))

    h2, as2, ad2 = _tc2(o1[0, :N, :64], o1[1, :N, :64],
                        o1[0, :N, 64:], o1[1, :N, 64:],
                        eas1, h1, W2, att_src2.reshape(1, 64),
                        att_dst2.reshape(1, 64), b1.reshape(1, 64))

    cpad = ((0, NP - N), (0, 7))
    o2 = _sc_layer2(jnp.pad(h2, rpad), jnp.pad(as2, cpad),
                    jnp.pad(ad2, cpad), srcp, dstp,
                    jnp.zeros((STRIPE, 72), _f32))

    return _tc3(o2[0, :N, :64], o2[1, :N, :64],
                o2[0, :N, 64:65], o2[1, :N, 64:65],
                as2, ad2, h2, b2.reshape(1, 64))


# speedup vs baseline: 55.8969x; 1.0249x over previous
import functools

import jax
import jax.numpy as jnp
from jax import lax
from jax.experimental import pallas as pl
from jax.experimental.pallas import tpu as pltpu
from jax.experimental.pallas import tpu_sc as plsc

N = 10000
D_IN = 256
OUT = 64
E = 160000

NC = 2
NS = 16
CH = 128
CHUNKS = 40
EP = NC * NS * CHUNKS * CH
NP = 10112
STRIPE = NP // NS

BLK = 1000
GRID = N // BLK

_f32 = jnp.float32

_SC_PARAMS = pltpu.CompilerParams(
    needs_layout_passes=False,
    use_tc_tiling_on_sc=False,
)


def _sel_mat(rows, cols):
    if rows < cols:
        return (lax.broadcasted_iota(jnp.int32, (rows, cols), 0)
                == lax.broadcasted_iota(jnp.int32, (rows, cols), 1)
                // (cols // rows)).astype(_f32)
    else:
        return (lax.broadcasted_iota(jnp.int32, (rows, cols), 0)
                // (rows // cols)
                == lax.broadcasted_iota(jnp.int32, (rows, cols), 1)
                ).astype(_f32)


def _tc1_body(x_ref, w1_ref, ats_ref, atd_ref, h_ref, as_ref, ad_ref,
              eas_ref):
    h = jnp.dot(x_ref[...], w1_ref[...], preferred_element_type=_f32)
    h_ref[...] = h
    sel = _sel_mat(64, 8)
    a_s = jnp.dot(h * ats_ref[...], sel, preferred_element_type=_f32)
    a_d = jnp.dot(h * atd_ref[...], sel, preferred_element_type=_f32)
    as_ref[...] = a_s
    ad_ref[...] = a_d
    al = a_s + a_d
    eas_ref[...] = jnp.exp(jnp.where(al >= 0, al, 0.2 * al))


def _tc1(x, w1, ats, atd):
    return pl.pallas_call(
        _tc1_body,
        grid=(GRID,),
        in_specs=[
            pl.BlockSpec((BLK, D_IN), lambda i: (i, 0)),
            pl.BlockSpec((D_IN, 64), lambda i: (0, 0)),
            pl.BlockSpec((1, 64), lambda i: (0, 0)),
            pl.BlockSpec((1, 64), lambda i: (0, 0)),
        ],
        out_specs=[
            pl.BlockSpec((BLK, 64), lambda i: (i, 0)),
            pl.BlockSpec((BLK, 8), lambda i: (i, 0)),
            pl.BlockSpec((BLK, 8), lambda i: (i, 0)),
            pl.BlockSpec((BLK, 8), lambda i: (i, 0)),
        ],
        out_shape=[
            jax.ShapeDtypeStruct((N, 64), _f32),
            jax.ShapeDtypeStruct((N, 8), _f32),
            jax.ShapeDtypeStruct((N, 8), _f32),
            jax.ShapeDtypeStruct((N, 8), _f32),
        ],
    )(x, w1, ats, atd)


def _tc2_body(m0_ref, m1_ref, s0_ref, s1_ref, eas_ref, h1_ref, w2_ref,
              at2s_ref, at2d_ref, b1_ref, h2_ref, as2_ref, ad2_ref):
    sel = _sel_mat(8, 64)
    eas = eas_ref[...]
    s64 = jnp.dot(s0_ref[...] + s1_ref[...] + eas, sel,
                  preferred_element_type=_f32)
    num = (m0_ref[...] + m1_ref[...]
           + h1_ref[...] * jnp.dot(eas, sel, preferred_element_type=_f32))
    o1 = num / (s64 + 1e-16) + b1_ref[...]
    h1p = jnp.where(o1 > 0, o1, jnp.exp(jnp.minimum(o1, 0.0)) - 1.0)
    h2 = jnp.dot(h1p, w2_ref[...], preferred_element_type=_f32)
    h2_ref[...] = h2
    as2_ref[...] = jnp.sum(h2 * at2s_ref[...], axis=1, keepdims=True)
    ad2_ref[...] = jnp.sum(h2 * at2d_ref[...], axis=1, keepdims=True)


def _tc2(m0, m1, s0, s1, eas, h1, w2, at2s, at2d, b1):
    return pl.pallas_call(
        _tc2_body,
        grid=(GRID,),
        in_specs=[
            pl.BlockSpec((BLK, 64), lambda i: (i, 0)),
            pl.BlockSpec((BLK, 64), lambda i: (i, 0)),
            pl.BlockSpec((BLK, 8), lambda i: (i, 0)),
            pl.BlockSpec((BLK, 8), lambda i: (i, 0)),
            pl.BlockSpec((BLK, 8), lambda i: (i, 0)),
            pl.BlockSpec((BLK, 64), lambda i: (i, 0)),
            pl.BlockSpec((64, 64), lambda i: (0, 0)),
            pl.BlockSpec((1, 64), lambda i: (0, 0)),
            pl.BlockSpec((1, 64), lambda i: (0, 0)),
            pl.BlockSpec((1, 64), lambda i: (0, 0)),
        ],
        out_specs=[
            pl.BlockSpec((BLK, 64), lambda i: (i, 0)),
            pl.BlockSpec((BLK, 1), lambda i: (i, 0)),
            pl.BlockSpec((BLK, 1), lambda i: (i, 0)),
        ],
        out_shape=[
            jax.ShapeDtypeStruct((N, 64), _f32),
            jax.ShapeDtypeStruct((N, 1), _f32),
            jax.ShapeDtypeStruct((N, 1), _f32),
        ],
    )(m0, m1, s0, s1, eas, h1, w2, at2s, at2d, b1)


def _tc3_body(m0_ref, m1_ref, s0_ref, s1_ref, as2_ref, ad2_ref, h2_ref,
              b2_ref, out_ref):
    al = as2_ref[...] + ad2_ref[...]
    eas2 = jnp.exp(jnp.where(al >= 0, al, 0.2 * al))
    s = s0_ref[...] + s1_ref[...] + eas2
    num = m0_ref[...] + m1_ref[...] + h2_ref[...] * eas2
    o = num / (s + 1e-16) + b2_ref[...]
    z = o - jnp.max(o, axis=1, keepdims=True)
    out_ref[...] = z - jnp.log(jnp.sum(jnp.exp(z), axis=1, keepdims=True))


def _tc3(m0, m1, s0, s1, as2, ad2, h2, b2):
    return pl.pallas_call(
        _tc3_body,
        grid=(GRID,),
        in_specs=[
            pl.BlockSpec((BLK, 64), lambda i: (i, 0)),
            pl.BlockSpec((BLK, 64), lambda i: (i, 0)),
            pl.BlockSpec((BLK, 1), lambda i: (i, 0)),
            pl.BlockSpec((BLK, 1), lambda i: (i, 0)),
            pl.BlockSpec((BLK, 1), lambda i: (i, 0)),
            pl.BlockSpec((BLK, 1), lambda i: (i, 0)),
            pl.BlockSpec((BLK, 64), lambda i: (i, 0)),
            pl.BlockSpec((1, 64), lambda i: (0, 0)),
        ],
        out_specs=pl.BlockSpec((BLK, 64), lambda i: (i, 0)),
        out_shape=jax.ShapeDtypeStruct((N, 64), _f32),
    )(m0, m1, s0, s1, as2, ad2, h2, b2)


def _sc_edge_pass(nea):
    w = 72

    @functools.partial(
        pl.kernel,
        out_type=jax.ShapeDtypeStruct((NC, NP, w), _f32),
        mesh=plsc.VectorSubcoreMesh(core_axis_name="c", subcore_axis_name="s"),
        compiler_params=_SC_PARAMS,
        scratch_types=[
            pltpu.VMEM_SHARED((NP, w), _f32),
            pltpu.VMEM((2, CH), jnp.int32),
            pltpu.VMEM((2, CH), jnp.int32),
            pltpu.VMEM((2, CH, 8), _f32),
            pltpu.VMEM((2, CH, 8), _f32),
            pltpu.VMEM((2, CH, 64), _f32),
            pltpu.VMEM((2, CH, w), _f32),
            pltpu.VMEM((2, CH), jnp.int32),
            pltpu.SemaphoreType.DMA,
            pltpu.SemaphoreType.DMA,
            pltpu.SemaphoreType.DMA,
            pltpu.SemaphoreType.DMA,
            pltpu.SemaphoreType.DMA,
            pltpu.SemaphoreType.DMA,
            pltpu.SemaphoreType.DMA,
            pltpu.SemaphoreType.DMA,
        ],
    )
    def edge_pass(h_hbm, as_hbm, ad_hbm, src_hbm, dst_hbm, z_hbm, out_hbm,
                  acc_sh, src_v, dst_v, asr_v, adr_v, hr_v, msg_v, dsc_v,
                  *sems):
        cid = lax.axis_index("c")
        sid = lax.axis_index("s")
        r0 = sid * STRIPE
        pltpu.sync_copy(z_hbm, acc_sh.at[pl.ds(r0, STRIPE)])
        plsc.subcore_barrier()

        i16 = lax.broadcasted_iota(jnp.int32, (16,), 0)
        p8 = i16 // 8
        e8 = i16 - 8 * p8
        base0 = (cid * NS + sid) * (CHUNKS * CH)

        if nea != 8:
            for b0 in range(2):
                def z_body(e):
                    plsc.store_scatter(msg_v.at[b0], [i16 * 0 + e, 56 + i16],
                                       jnp.zeros((16,), _f32))
                plsc.parallel_loop(0, CH, 1, unroll=4)(z_body)

        def prefetch(c, b):
            base = base0 + c * CH
            pltpu.sync_copy(src_hbm.at[pl.ds(base, CH)], src_v.at[b])
            pltpu.sync_copy(dst_hbm.at[pl.ds(base, CH)], dst_v.at[b])
            pltpu.async_copy(as_hbm.at[src_v.at[b]], asr_v.at[b], sems[3 * b])
            pltpu.async_copy(ad_hbm.at[dst_v.at[b]], adr_v.at[b],
                             sems[3 * b + 1])
            pltpu.async_copy(h_hbm.at[src_v.at[b]], hr_v.at[b],
                             sems[3 * b + 2])

        def compute(b, drain_pred):
            asr_b, adr_b, hr_b = asr_v.at[b], adr_v.at[b], hr_v.at[b]
            msg_b = msg_v.at[b]
            pltpu.make_async_copy(as_hbm.at[src_v.at[b]], asr_b,
                                  sems[3 * b]).wait()
            pltpu.make_async_copy(ad_hbm.at[dst_v.at[b]], adr_b,
                                  sems[3 * b + 1]).wait()

            @pl.when(drain_pred)
            def _():
                pltpu.make_async_copy(msg_b, acc_sh.at[dsc_v.at[b]],
                                      sems[6 + b]).wait()

            if nea == 8:
                def ea_body(v):
                    rows = 2 * v + p8
                    a = (plsc.load_gather(asr_b, [rows, e8])
                         + plsc.load_gather(adr_b, [rows, e8]))
                    a = jnp.where(a >= 0, a, 0.2 * a)
                    plsc.store_scatter(msg_b, [rows, 64 + e8], jnp.exp(a))
                plsc.parallel_loop(0, CH // 2, 1, unroll=4)(ea_body)
            else:
                z16 = i16 * 0
                def ea_body(v):
                    rows = 16 * v + i16
                    a = (plsc.load_gather(asr_b, [rows, z16])
                         + plsc.load_gather(adr_b, [rows, z16]))
                    a = jnp.where(a >= 0, a, 0.2 * a)
                    plsc.store_scatter(msg_b, [rows, z16 + 64], jnp.exp(a))
                plsc.parallel_loop(0, CH // 16, 1, unroll=2)(ea_body)

            pltpu.make_async_copy(h_hbm.at[src_v.at[b]], hr_b,
                                  sems[3 * b + 2]).wait()

            if nea == 8:
                def mul_body(e):
                    erow = i16 * 0 + e
                    for k in range(4):
                        mult = plsc.load_gather(msg_b, [erow, 64 + 2 * k + p8])
                        hv = plsc.load_gather(hr_b, [erow, 16 * k + i16])
                        plsc.store_scatter(msg_b, [erow, 16 * k + i16],
                                           hv * mult)
                plsc.parallel_loop(0, CH, 1, unroll=4)(mul_body)
            else:
                z16 = i16 * 0
                def mul_body(e):
                    erow = i16 * 0 + e
                    mult = plsc.load_gather(msg_b, [erow, z16 + 64])
                    for k in range(4):
                        hv = plsc.load_gather(hr_b, [erow, 16 * k + i16])
                        plsc.store_scatter(msg_b, [erow, 16 * k + i16],
                                           hv * mult)
                plsc.parallel_loop(0, CH, 1, unroll=4)(mul_body)

            for v in range(CH // 16):
                dsc_v[b, pl.ds(16 * v, 16)] = dst_v[b, pl.ds(16 * v, 16)]
            pltpu.async_copy(msg_b, acc_sh.at[dsc_v.at[b]], sems[6 + b],
                             add=True)

        prefetch(0, 0)

        def chunk_body(i, carry):
            j = 2 * i
            prefetch(j + 1, 1)
            compute(0, i > 0)

            @pl.when(j + 2 < CHUNKS)
            def _():
                prefetch(j + 2, 0)
            compute(1, i > 0)
            return carry

        lax.fori_loop(0, CHUNKS // 2, chunk_body, 0)
        pltpu.make_async_copy(msg_v.at[0], acc_sh.at[dsc_v.at[0]],
                              sems[6]).wait()
        pltpu.make_async_copy(msg_v.at[1], acc_sh.at[dsc_v.at[1]],
                              sems[7]).wait()
        plsc.subcore_barrier()
        pltpu.sync_copy(acc_sh.at[pl.ds(r0, STRIPE)],
                        out_hbm.at[cid, pl.ds(r0, STRIPE)])

    return edge_pass


_sc_layer1 = _sc_edge_pass(8)
_sc_layer2 = _sc_edge_pass(1)


def kernel(x, edge_index, W1, att_src1, att_dst1, b1, W2, att_src2,
           att_dst2, b2):
    src = edge_index[0]
    dst = edge_index[1]
    pad = jnp.full((EP - E,), N, jnp.int32)
    srcp = jnp.concatenate([src, pad])
    dstp = jnp.concatenate([dst, pad])

    h1, as1, ad1, eas1 = _tc1(x, W1, att_src1.reshape(1, 64),
                              att_dst1.reshape(1, 64))

    rpad = ((0, NP - N), (0, 0))
    o1 = _sc_layer1(jnp.pad(h1, rpad), jnp.pad(as1, rpad),
                    jnp.pad(ad1, rpad), srcp, dstp,
                    jnp.zeros((STRIPE, 72), _f32))

    h2, as2, ad2 = _tc2(o1[0, :N, :64], o1[1, :N, :64],
                        o1[0, :N, 64:], o1[1, :N, 64:],
                        eas1, h1, W2, att_src2.reshape(1, 64),
                        att_dst2.reshape(1, 64), b1.reshape(1, 64))

    cpad = ((0, NP - N), (0, 7))
    o2 = _sc_layer2(jnp.pad(h2, rpad), jnp.pad(as2, cpad),
                    jnp.pad(ad2, cpad), srcp, dstp,
                    jnp.zeros((STRIPE, 72), _f32))

    return _tc3(o2[0, :N, :64], o2[1, :N, :64],
                o2[0, :N, 64:65], o2[1, :N, 64:65],
                as2, ad2, h2, b2.reshape(1, 64))


# speedup vs baseline: 62.8068x; 1.1236x over previous
import functools

import jax
import jax.numpy as jnp
from jax import lax
from jax.experimental import pallas as pl
from jax.experimental.pallas import tpu as pltpu
from jax.experimental.pallas import tpu_sc as plsc

N = 10000
D_IN = 256
OUT = 64
E = 160000

NC = 2
NS = 16
CH = 128
CHUNKS = 40
CH0 = 52
CH1 = 2 * CHUNKS - CH0
EP = NC * NS * CHUNKS * CH
NP = 10112
STRIPE = NP // NS

BLK = 1000
GRID = N // BLK

_f32 = jnp.float32

_SC_PARAMS = pltpu.CompilerParams(
    needs_layout_passes=False,
    use_tc_tiling_on_sc=False,
)


def _sel_mat(rows, cols):
    if rows < cols:
        return (lax.broadcasted_iota(jnp.int32, (rows, cols), 0)
                == lax.broadcasted_iota(jnp.int32, (rows, cols), 1)
                // (cols // rows)).astype(_f32)
    else:
        return (lax.broadcasted_iota(jnp.int32, (rows, cols), 0)
                // (rows // cols)
                == lax.broadcasted_iota(jnp.int32, (rows, cols), 1)
                ).astype(_f32)


def _tc1_body(x_ref, w1_ref, ats_ref, atd_ref, h_ref, as_ref, ad_ref,
              eas_ref):
    h = jnp.dot(x_ref[...], w1_ref[...], preferred_element_type=_f32)
    h_ref[...] = h
    sel = _sel_mat(64, 8)
    a_s = jnp.dot(h * ats_ref[...], sel, preferred_element_type=_f32)
    a_d = jnp.dot(h * atd_ref[...], sel, preferred_element_type=_f32)
    as_ref[...] = a_s
    ad_ref[...] = a_d
    al = a_s + a_d
    eas_ref[...] = jnp.exp(jnp.where(al >= 0, al, 0.2 * al))


def _tc1(x, w1, ats, atd):
    return pl.pallas_call(
        _tc1_body,
        grid=(GRID,),
        in_specs=[
            pl.BlockSpec((BLK, D_IN), lambda i: (i, 0)),
            pl.BlockSpec((D_IN, 64), lambda i: (0, 0)),
            pl.BlockSpec((1, 64), lambda i: (0, 0)),
            pl.BlockSpec((1, 64), lambda i: (0, 0)),
        ],
        out_specs=[
            pl.BlockSpec((BLK, 64), lambda i: (i, 0)),
            pl.BlockSpec((BLK, 8), lambda i: (i, 0)),
            pl.BlockSpec((BLK, 8), lambda i: (i, 0)),
            pl.BlockSpec((BLK, 8), lambda i: (i, 0)),
        ],
        out_shape=[
            jax.ShapeDtypeStruct((N, 64), _f32),
            jax.ShapeDtypeStruct((N, 8), _f32),
            jax.ShapeDtypeStruct((N, 8), _f32),
            jax.ShapeDtypeStruct((N, 8), _f32),
        ],
    )(x, w1, ats, atd)


def _tc2_body(m0_ref, m1_ref, s0_ref, s1_ref, eas_ref, h1_ref, w2_ref,
              at2s_ref, at2d_ref, b1_ref, h2_ref, as2_ref, ad2_ref):
    sel = _sel_mat(8, 64)
    eas = eas_ref[...]
    s64 = jnp.dot(s0_ref[...] + s1_ref[...] + eas, sel,
                  preferred_element_type=_f32)
    num = (m0_ref[...] + m1_ref[...]
           + h1_ref[...] * jnp.dot(eas, sel, preferred_element_type=_f32))
    o1 = num / (s64 + 1e-16) + b1_ref[...]
    h1p = jnp.where(o1 > 0, o1, jnp.exp(jnp.minimum(o1, 0.0)) - 1.0)
    h2 = jnp.dot(h1p, w2_ref[...], preferred_element_type=_f32)
    h2_ref[...] = h2
    as2_ref[...] = jnp.sum(h2 * at2s_ref[...], axis=1, keepdims=True)
    ad2_ref[...] = jnp.sum(h2 * at2d_ref[...], axis=1, keepdims=True)


def _tc2(m0, m1, s0, s1, eas, h1, w2, at2s, at2d, b1):
    return pl.pallas_call(
        _tc2_body,
        grid=(GRID,),
        in_specs=[
            pl.BlockSpec((BLK, 64), lambda i: (i, 0)),
            pl.BlockSpec((BLK, 64), lambda i: (i, 0)),
            pl.BlockSpec((BLK, 8), lambda i: (i, 0)),
            pl.BlockSpec((BLK, 8), lambda i: (i, 0)),
            pl.BlockSpec((BLK, 8), lambda i: (i, 0)),
            pl.BlockSpec((BLK, 64), lambda i: (i, 0)),
            pl.BlockSpec((64, 64), lambda i: (0, 0)),
            pl.BlockSpec((1, 64), lambda i: (0, 0)),
            pl.BlockSpec((1, 64), lambda i: (0, 0)),
            pl.BlockSpec((1, 64), lambda i: (0, 0)),
        ],
        out_specs=[
            pl.BlockSpec((BLK, 64), lambda i: (i, 0)),
            pl.BlockSpec((BLK, 1), lambda i: (i, 0)),
            pl.BlockSpec((BLK, 1), lambda i: (i, 0)),
        ],
        out_shape=[
            jax.ShapeDtypeStruct((N, 64), _f32),
            jax.ShapeDtypeStruct((N, 1), _f32),
            jax.ShapeDtypeStruct((N, 1), _f32),
        ],
    )(m0, m1, s0, s1, eas, h1, w2, at2s, at2d, b1)


def _tc3_body(m0_ref, m1_ref, s0_ref, s1_ref, as2_ref, ad2_ref, h2_ref,
              b2_ref, out_ref):
    al = as2_ref[...] + ad2_ref[...]
    eas2 = jnp.exp(jnp.where(al >= 0, al, 0.2 * al))
    s = s0_ref[...] + s1_ref[...] + eas2
    num = m0_ref[...] + m1_ref[...] + h2_ref[...] * eas2
    o = num / (s + 1e-16) + b2_ref[...]
    z = o - jnp.max(o, axis=1, keepdims=True)
    out_ref[...] = z - jnp.log(jnp.sum(jnp.exp(z), axis=1, keepdims=True))


def _tc3(m0, m1, s0, s1, as2, ad2, h2, b2):
    return pl.pallas_call(
        _tc3_body,
        grid=(GRID,),
        in_specs=[
            pl.BlockSpec((BLK, 64), lambda i: (i, 0)),
            pl.BlockSpec((BLK, 64), lambda i: (i, 0)),
            pl.BlockSpec((BLK, 1), lambda i: (i, 0)),
            pl.BlockSpec((BLK, 1), lambda i: (i, 0)),
            pl.BlockSpec((BLK, 1), lambda i: (i, 0)),
            pl.BlockSpec((BLK, 1), lambda i: (i, 0)),
            pl.BlockSpec((BLK, 64), lambda i: (i, 0)),
            pl.BlockSpec((1, 64), lambda i: (0, 0)),
        ],
        out_specs=pl.BlockSpec((BLK, 64), lambda i: (i, 0)),
        out_shape=jax.ShapeDtypeStruct((N, 64), _f32),
    )(m0, m1, s0, s1, as2, ad2, h2, b2)


def _sc_edge_pass(nea):
    w = 72

    @functools.partial(
        pl.kernel,
        out_type=jax.ShapeDtypeStruct((NC, NP, w), _f32),
        mesh=plsc.VectorSubcoreMesh(core_axis_name="c", subcore_axis_name="s"),
        compiler_params=_SC_PARAMS,
        scratch_types=[
            pltpu.VMEM_SHARED((NP, w), _f32),
            pltpu.VMEM((2, CH), jnp.int32),
            pltpu.VMEM((2, CH), jnp.int32),
            pltpu.VMEM((2, CH, 8), _f32),
            pltpu.VMEM((2, CH, 8), _f32),
            pltpu.VMEM((2, CH, 64), _f32),
            pltpu.VMEM((2, CH, w), _f32),
            pltpu.VMEM((2, CH), jnp.int32),
            pltpu.SemaphoreType.DMA,
            pltpu.SemaphoreType.DMA,
            pltpu.SemaphoreType.DMA,
            pltpu.SemaphoreType.DMA,
            pltpu.SemaphoreType.DMA,
            pltpu.SemaphoreType.DMA,
            pltpu.SemaphoreType.DMA,
            pltpu.SemaphoreType.DMA,
        ],
    )
    def edge_pass(h_hbm, as_hbm, ad_hbm, src_hbm, dst_hbm, z_hbm, out_hbm,
                  acc_sh, src_v, dst_v, asr_v, adr_v, hr_v, msg_v, dsc_v,
                  *sems):
        cid = lax.axis_index("c")
        sid = lax.axis_index("s")
        r0 = sid * STRIPE
        pltpu.sync_copy(z_hbm, acc_sh.at[pl.ds(r0, STRIPE)])
        plsc.subcore_barrier()

        i16 = lax.broadcasted_iota(jnp.int32, (16,), 0)
        p8 = i16 // 8
        e8 = i16 - 8 * p8
        nch = jnp.where(cid == 0, CH0, CH1)
        base0 = jnp.where(cid == 0, sid * (CH0 * CH),
                          NS * CH0 * CH + sid * (CH1 * CH))

        if nea != 8:
            for b0 in range(2):
                def z_body(e):
                    plsc.store_scatter(msg_v.at[b0], [i16 * 0 + e, 56 + i16],
                                       jnp.zeros((16,), _f32))
                plsc.parallel_loop(0, CH, 1, unroll=4)(z_body)

        def prefetch(c, b):
            base = base0 + c * CH
            pltpu.sync_copy(src_hbm.at[pl.ds(base, CH)], src_v.at[b])
            pltpu.sync_copy(dst_hbm.at[pl.ds(base, CH)], dst_v.at[b])
            pltpu.async_copy(as_hbm.at[src_v.at[b]], asr_v.at[b], sems[3 * b])
            pltpu.async_copy(ad_hbm.at[dst_v.at[b]], adr_v.at[b],
                             sems[3 * b + 1])
            pltpu.async_copy(h_hbm.at[src_v.at[b]], hr_v.at[b],
                             sems[3 * b + 2])

        def compute(b, drain_pred):
            asr_b, adr_b, hr_b = asr_v.at[b], adr_v.at[b], hr_v.at[b]
            msg_b = msg_v.at[b]
            pltpu.make_async_copy(as_hbm.at[src_v.at[b]], asr_b,
                                  sems[3 * b]).wait()
            pltpu.make_async_copy(ad_hbm.at[dst_v.at[b]], adr_b,
                                  sems[3 * b + 1]).wait()

            @pl.when(drain_pred)
            def _():
                pltpu.make_async_copy(msg_b, acc_sh.at[dsc_v.at[b]],
                                      sems[6 + b]).wait()

            if nea == 8:
                def ea_body(v):
                    rows = 2 * v + p8
                    a = (plsc.load_gather(asr_b, [rows, e8])
                         + plsc.load_gather(adr_b, [rows, e8]))
                    a = jnp.where(a >= 0, a, 0.2 * a)
                    plsc.store_scatter(msg_b, [rows, 64 + e8], jnp.exp(a))
                plsc.parallel_loop(0, CH // 2, 1, unroll=4)(ea_body)
            else:
                z16 = i16 * 0
                def ea_body(v):
                    rows = 16 * v + i16
                    a = (plsc.load_gather(asr_b, [rows, z16])
                         + plsc.load_gather(adr_b, [rows, z16]))
                    a = jnp.where(a >= 0, a, 0.2 * a)
                    plsc.store_scatter(msg_b, [rows, z16 + 64], jnp.exp(a))
                plsc.parallel_loop(0, CH // 16, 1, unroll=2)(ea_body)

            pltpu.make_async_copy(h_hbm.at[src_v.at[b]], hr_b,
                                  sems[3 * b + 2]).wait()

            if nea == 8:
                def mul_body(e):
                    erow = i16 * 0 + e
                    for k in range(4):
                        mult = plsc.load_gather(msg_b, [erow, 64 + 2 * k + p8])
                        hv = plsc.load_gather(hr_b, [erow, 16 * k + i16])
                        plsc.store_scatter(msg_b, [erow, 16 * k + i16],
                                           hv * mult)
                plsc.parallel_loop(0, CH, 1, unroll=4)(mul_body)
            else:
                z16 = i16 * 0
                def mul_body(e):
                    erow = i16 * 0 + e
                    mult = plsc.load_gather(msg_b, [erow, z16 + 64])
                    for k in range(4):
                        hv = plsc.load_gather(hr_b, [erow, 16 * k + i16])
                        plsc.store_scatter(msg_b, [erow, 16 * k + i16],
                                           hv * mult)
                plsc.parallel_loop(0, CH, 1, unroll=4)(mul_body)

            for v in range(CH // 16):
                dsc_v[b, pl.ds(16 * v, 16)] = dst_v[b, pl.ds(16 * v, 16)]
            pltpu.async_copy(msg_b, acc_sh.at[dsc_v.at[b]], sems[6 + b],
                             add=True)

        prefetch(0, 0)

        def chunk_body(i, carry):
            j = 2 * i
            prefetch(j + 1, 1)
            compute(0, i > 0)

            @pl.when(j + 2 < nch)
            def _():
                prefetch(j + 2, 0)
            compute(1, i > 0)
            return carry

        lax.fori_loop(0, nch // 2, chunk_body, 0)
        pltpu.make_async_copy(msg_v.at[0], acc_sh.at[dsc_v.at[0]],
                              sems[6]).wait()
        pltpu.make_async_copy(msg_v.at[1], acc_sh.at[dsc_v.at[1]],
                              sems[7]).wait()
        plsc.subcore_barrier()
        pltpu.sync_copy(acc_sh.at[pl.ds(r0, STRIPE)],
                        out_hbm.at[cid, pl.ds(r0, STRIPE)])

    return edge_pass


_sc_layer1 = _sc_edge_pass(8)
_sc_layer2 = _sc_edge_pass(1)


def kernel(x, edge_index, W1, att_src1, att_dst1, b1, W2, att_src2,
           att_dst2, b2):
    src = edge_index[0]
    dst = edge_index[1]
    pad = jnp.full((EP - E,), N, jnp.int32)
    srcp = jnp.concatenate([src, pad])
    dstp = jnp.concatenate([dst, pad])

    h1, as1, ad1, eas1 = _tc1(x, W1, att_src1.reshape(1, 64),
                              att_dst1.reshape(1, 64))

    rpad = ((0, NP - N), (0, 0))
    o1 = _sc_layer1(jnp.pad(h1, rpad), jnp.pad(as1, rpad),
                    jnp.pad(ad1, rpad), srcp, dstp,
                    jnp.zeros((STRIPE, 72), _f32))

    h2, as2, ad2 = _tc2(o1[0, :N, :64], o1[1, :N, :64],
                        o1[0, :N, 64:], o1[1, :N, 64:],
                        eas1, h1, W2, att_src2.reshape(1, 64),
                        att_dst2.reshape(1, 64), b1.reshape(1, 64))

    cpad = ((0, NP - N), (0, 7))
    o2 = _sc_layer2(jnp.pad(h2, rpad), jnp.pad(as2, cpad),
                    jnp.pad(ad2, cpad), srcp, dstp,
                    jnp.zeros((STRIPE, 72), _f32))

    return _tc3(o2[0, :N, :64], o2[1, :N, :64],
                o2[0, :N, 64:65], o2[1, :N, 64:65],
                as2, ad2, h2, b2.reshape(1, 64))


# speedup vs baseline: 63.3617x; 1.0088x over previous
import functools

import jax
import jax.numpy as jnp
from jax import lax
from jax.experimental import pallas as pl
from jax.experimental.pallas import tpu as pltpu
from jax.experimental.pallas import tpu_sc as plsc

N = 10000
D_IN = 256
OUT = 64
E = 160000

NC = 2
NS = 16
CH = 128
CHUNKS = 40
CH0 = 54
CH1 = 2 * CHUNKS - CH0
EP = NC * NS * CHUNKS * CH
NP = 10112
STRIPE = NP // NS

BLK = 1000
GRID = N // BLK

_f32 = jnp.float32

_SC_PARAMS = pltpu.CompilerParams(
    needs_layout_passes=False,
    use_tc_tiling_on_sc=False,
)


def _sel_mat(rows, cols):
    if rows < cols:
        return (lax.broadcasted_iota(jnp.int32, (rows, cols), 0)
                == lax.broadcasted_iota(jnp.int32, (rows, cols), 1)
                // (cols // rows)).astype(_f32)
    else:
        return (lax.broadcasted_iota(jnp.int32, (rows, cols), 0)
                // (rows // cols)
                == lax.broadcasted_iota(jnp.int32, (rows, cols), 1)
                ).astype(_f32)


def _tc1_body(x_ref, w1_ref, ats_ref, atd_ref, h_ref, as_ref, ad_ref,
              eas_ref):
    h = jnp.dot(x_ref[...], w1_ref[...], preferred_element_type=_f32)
    h_ref[...] = h
    sel = _sel_mat(64, 8)
    a_s = jnp.dot(h * ats_ref[...], sel, preferred_element_type=_f32)
    a_d = jnp.dot(h * atd_ref[...], sel, preferred_element_type=_f32)
    as_ref[...] = a_s
    ad_ref[...] = a_d
    al = a_s + a_d
    eas_ref[...] = jnp.exp(jnp.where(al >= 0, al, 0.2 * al))


def _tc1(x, w1, ats, atd):
    return pl.pallas_call(
        _tc1_body,
        grid=(GRID,),
        in_specs=[
            pl.BlockSpec((BLK, D_IN), lambda i: (i, 0)),
            pl.BlockSpec((D_IN, 64), lambda i: (0, 0)),
            pl.BlockSpec((1, 64), lambda i: (0, 0)),
            pl.BlockSpec((1, 64), lambda i: (0, 0)),
        ],
        out_specs=[
            pl.BlockSpec((BLK, 64), lambda i: (i, 0)),
            pl.BlockSpec((BLK, 8), lambda i: (i, 0)),
            pl.BlockSpec((BLK, 8), lambda i: (i, 0)),
            pl.BlockSpec((BLK, 8), lambda i: (i, 0)),
        ],
        out_shape=[
            jax.ShapeDtypeStruct((N, 64), _f32),
            jax.ShapeDtypeStruct((N, 8), _f32),
            jax.ShapeDtypeStruct((N, 8), _f32),
            jax.ShapeDtypeStruct((N, 8), _f32),
        ],
    )(x, w1, ats, atd)


def _tc2_body(m0_ref, m1_ref, s0_ref, s1_ref, eas_ref, h1_ref, w2_ref,
              at2s_ref, at2d_ref, b1_ref, h2_ref, as2_ref, ad2_ref):
    sel = _sel_mat(8, 64)
    eas = eas_ref[...]
    s64 = jnp.dot(s0_ref[...] + s1_ref[...] + eas, sel,
                  preferred_element_type=_f32)
    num = (m0_ref[...] + m1_ref[...]
           + h1_ref[...] * jnp.dot(eas, sel, preferred_element_type=_f32))
    o1 = num / (s64 + 1e-16) + b1_ref[...]
    h1p = jnp.where(o1 > 0, o1, jnp.exp(jnp.minimum(o1, 0.0)) - 1.0)
    h2 = jnp.dot(h1p, w2_ref[...], preferred_element_type=_f32)
    h2_ref[...] = h2
    as2_ref[...] = jnp.sum(h2 * at2s_ref[...], axis=1, keepdims=True)
    ad2_ref[...] = jnp.sum(h2 * at2d_ref[...], axis=1, keepdims=True)


def _tc2(m0, m1, s0, s1, eas, h1, w2, at2s, at2d, b1):
    return pl.pallas_call(
        _tc2_body,
        grid=(GRID,),
        in_specs=[
            pl.BlockSpec((BLK, 64), lambda i: (i, 0)),
            pl.BlockSpec((BLK, 64), lambda i: (i, 0)),
            pl.BlockSpec((BLK, 8), lambda i: (i, 0)),
            pl.BlockSpec((BLK, 8), lambda i: (i, 0)),
            pl.BlockSpec((BLK, 8), lambda i: (i, 0)),
            pl.BlockSpec((BLK, 64), lambda i: (i, 0)),
            pl.BlockSpec((64, 64), lambda i: (0, 0)),
            pl.BlockSpec((1, 64), lambda i: (0, 0)),
            pl.BlockSpec((1, 64), lambda i: (0, 0)),
            pl.BlockSpec((1, 64), lambda i: (0, 0)),
        ],
        out_specs=[
            pl.BlockSpec((BLK, 64), lambda i: (i, 0)),
            pl.BlockSpec((BLK, 1), lambda i: (i, 0)),
            pl.BlockSpec((BLK, 1), lambda i: (i, 0)),
        ],
        out_shape=[
            jax.ShapeDtypeStruct((N, 64), _f32),
            jax.ShapeDtypeStruct((N, 1), _f32),
            jax.ShapeDtypeStruct((N, 1), _f32),
        ],
    )(m0, m1, s0, s1, eas, h1, w2, at2s, at2d, b1)


def _tc3_body(m0_ref, m1_ref, s0_ref, s1_ref, as2_ref, ad2_ref, h2_ref,
              b2_ref, out_ref):
    al = as2_ref[...] + ad2_ref[...]
    eas2 = jnp.exp(jnp.where(al >= 0, al, 0.2 * al))
    s = s0_ref[...] + s1_ref[...] + eas2
    num = m0_ref[...] + m1_ref[...] + h2_ref[...] * eas2
    o = num / (s + 1e-16) + b2_ref[...]
    z = o - jnp.max(o, axis=1, keepdims=True)
    out_ref[...] = z - jnp.log(jnp.sum(jnp.exp(z), axis=1, keepdims=True))


def _tc3(m0, m1, s0, s1, as2, ad2, h2, b2):
    return pl.pallas_call(
        _tc3_body,
        grid=(GRID,),
        in_specs=[
            pl.BlockSpec((BLK, 64), lambda i: (i, 0)),
            pl.BlockSpec((BLK, 64), lambda i: (i, 0)),
            pl.BlockSpec((BLK, 1), lambda i: (i, 0)),
            pl.BlockSpec((BLK, 1), lambda i: (i, 0)),
            pl.BlockSpec((BLK, 1), lambda i: (i, 0)),
            pl.BlockSpec((BLK, 1), lambda i: (i, 0)),
            pl.BlockSpec((BLK, 64), lambda i: (i, 0)),
            pl.BlockSpec((1, 64), lambda i: (0, 0)),
        ],
        out_specs=pl.BlockSpec((BLK, 64), lambda i: (i, 0)),
        out_shape=jax.ShapeDtypeStruct((N, 64), _f32),
    )(m0, m1, s0, s1, as2, ad2, h2, b2)


def _sc_edge_pass(nea):
    w = 72

    @functools.partial(
        pl.kernel,
        out_type=jax.ShapeDtypeStruct((NC, NP, w), _f32),
        mesh=plsc.VectorSubcoreMesh(core_axis_name="c", subcore_axis_name="s"),
        compiler_params=_SC_PARAMS,
        scratch_types=[
            pltpu.VMEM_SHARED((NP, w), _f32),
            pltpu.VMEM((2, CH), jnp.int32),
            pltpu.VMEM((2, CH), jnp.int32),
            pltpu.VMEM((2, CH, 8), _f32),
            pltpu.VMEM((2, CH, 8), _f32),
            pltpu.VMEM((2, CH, 64), _f32),
            pltpu.VMEM((2, CH, w), _f32),
            pltpu.VMEM((2, CH), jnp.int32),
            pltpu.SemaphoreType.DMA,
            pltpu.SemaphoreType.DMA,
            pltpu.SemaphoreType.DMA,
            pltpu.SemaphoreType.DMA,
            pltpu.SemaphoreType.DMA,
            pltpu.SemaphoreType.DMA,
            pltpu.SemaphoreType.DMA,
            pltpu.SemaphoreType.DMA,
        ],
    )
    def edge_pass(h_hbm, as_hbm, ad_hbm, src_hbm, dst_hbm, z_hbm, out_hbm,
                  acc_sh, src_v, dst_v, asr_v, adr_v, hr_v, msg_v, dsc_v,
                  *sems):
        cid = lax.axis_index("c")
        sid = lax.axis_index("s")
        r0 = sid * STRIPE
        pltpu.sync_copy(z_hbm, acc_sh.at[pl.ds(r0, STRIPE)])
        plsc.subcore_barrier()

        i16 = lax.broadcasted_iota(jnp.int32, (16,), 0)
        p8 = i16 // 8
        e8 = i16 - 8 * p8
        nch = jnp.where(cid == 0, CH0, CH1)
        base0 = jnp.where(cid == 0, sid * (CH0 * CH),
                          NS * CH0 * CH + sid * (CH1 * CH))

        if nea != 8:
            for b0 in range(2):
                def z_body(e):
                    plsc.store_scatter(msg_v.at[b0], [i16 * 0 + e, 56 + i16],
                                       jnp.zeros((16,), _f32))
                plsc.parallel_loop(0, CH, 1, unroll=4)(z_body)

        def prefetch(c, b):
            base = base0 + c * CH
            pltpu.sync_copy(src_hbm.at[pl.ds(base, CH)], src_v.at[b])
            pltpu.sync_copy(dst_hbm.at[pl.ds(base, CH)], dst_v.at[b])
            pltpu.async_copy(as_hbm.at[src_v.at[b]], asr_v.at[b], sems[3 * b])
            pltpu.async_copy(ad_hbm.at[dst_v.at[b]], adr_v.at[b],
                             sems[3 * b + 1])
            pltpu.async_copy(h_hbm.at[src_v.at[b]], hr_v.at[b],
                             sems[3 * b + 2])

        def compute(b, drain_pred):
            asr_b, adr_b, hr_b = asr_v.at[b], adr_v.at[b], hr_v.at[b]
            msg_b = msg_v.at[b]
            pltpu.make_async_copy(as_hbm.at[src_v.at[b]], asr_b,
                                  sems[3 * b]).wait()
            pltpu.make_async_copy(ad_hbm.at[dst_v.at[b]], adr_b,
                                  sems[3 * b + 1]).wait()

            @pl.when(drain_pred)
            def _():
                pltpu.make_async_copy(msg_b, acc_sh.at[dsc_v.at[b]],
                                      sems[6 + b]).wait()

            if nea == 8:
                def ea_body(v):
                    rows = 2 * v + p8
                    a = (plsc.load_gather(asr_b, [rows, e8])
                         + plsc.load_gather(adr_b, [rows, e8]))
                    a = jnp.where(a >= 0, a, 0.2 * a)
                    plsc.store_scatter(msg_b, [rows, 64 + e8], jnp.exp(a))
                plsc.parallel_loop(0, CH // 2, 1, unroll=4)(ea_body)
            else:
                z16 = i16 * 0
                def ea_body(v):
                    rows = 16 * v + i16
                    a = (plsc.load_gather(asr_b, [rows, z16])
                         + plsc.load_gather(adr_b, [rows, z16]))
                    a = jnp.where(a >= 0, a, 0.2 * a)
                    plsc.store_scatter(msg_b, [rows, z16 + 64], jnp.exp(a))
                plsc.parallel_loop(0, CH // 16, 1, unroll=2)(ea_body)

            pltpu.make_async_copy(h_hbm.at[src_v.at[b]], hr_b,
                                  sems[3 * b + 2]).wait()

            if nea == 8:
                def mul_body(e):
                    erow = i16 * 0 + e
                    for k in range(4):
                        mult = plsc.load_gather(msg_b, [erow, 64 + 2 * k + p8])
                        hv = plsc.load_gather(hr_b, [erow, 16 * k + i16])
                        plsc.store_scatter(msg_b, [erow, 16 * k + i16],
                                           hv * mult)
                plsc.parallel_loop(0, CH, 1, unroll=4)(mul_body)
            else:
                z16 = i16 * 0
                def mul_body(e):
                    erow = i16 * 0 + e
                    mult = plsc.load_gather(msg_b, [erow, z16 + 64])
                    for k in range(4):
                        hv = plsc.load_gather(hr_b, [erow, 16 * k + i16])
                        plsc.store_scatter(msg_b, [erow, 16 * k + i16],
                                           hv * mult)
                plsc.parallel_loop(0, CH, 1, unroll=4)(mul_body)

            for v in range(CH // 16):
                dsc_v[b, pl.ds(16 * v, 16)] = dst_v[b, pl.ds(16 * v, 16)]
            pltpu.async_copy(msg_b, acc_sh.at[dsc_v.at[b]], sems[6 + b],
                             add=True)

        prefetch(0, 0)

        def chunk_body(i, carry):
            j = 2 * i
            prefetch(j + 1, 1)
            compute(0, i > 0)

            @pl.when(j + 2 < nch)
            def _():
                prefetch(j + 2, 0)
            compute(1, i > 0)
            return carry

        lax.fori_loop(0, nch // 2, chunk_body, 0)
        pltpu.make_async_copy(msg_v.at[0], acc_sh.at[dsc_v.at[0]],
                              sems[6]).wait()
        pltpu.make_async_copy(msg_v.at[1], acc_sh.at[dsc_v.at[1]],
                              sems[7]).wait()
        plsc.subcore_barrier()
        pltpu.sync_copy(acc_sh.at[pl.ds(r0, STRIPE)],
                        out_hbm.at[cid, pl.ds(r0, STRIPE)])

    return edge_pass


_sc_layer1 = _sc_edge_pass(8)
_sc_layer2 = _sc_edge_pass(1)


def kernel(x, edge_index, W1, att_src1, att_dst1, b1, W2, att_src2,
           att_dst2, b2):
    src = edge_index[0]
    dst = edge_index[1]
    pad = jnp.full((EP - E,), N, jnp.int32)
    srcp = jnp.concatenate([src, pad])
    dstp = jnp.concatenate([dst, pad])

    h1, as1, ad1, eas1 = _tc1(x, W1, att_src1.reshape(1, 64),
                              att_dst1.reshape(1, 64))

    rpad = ((0, NP - N), (0, 0))
    o1 = _sc_layer1(jnp.pad(h1, rpad), jnp.pad(as1, rpad),
                    jnp.pad(ad1, rpad), srcp, dstp,
                    jnp.zeros((STRIPE, 72), _f32))

    h2, as2, ad2 = _tc2(o1[0, :N, :64], o1[1, :N, :64],
                        o1[0, :N, 64:], o1[1, :N, 64:],
                        eas1, h1, W2, att_src2.reshape(1, 64),
                        att_dst2.reshape(1, 64), b1.reshape(1, 64))

    cpad = ((0, NP - N), (0, 7))
    o2 = _sc_layer2(jnp.pad(h2, rpad), jnp.pad(as2, cpad),
                    jnp.pad(ad2, cpad), srcp, dstp,
                    jnp.zeros((STRIPE, 72), _f32))

    return _tc3(o2[0, :N, :64], o2[1, :N, :64],
                o2[0, :N, 64:65], o2[1, :N, 64:65],
                as2, ad2, h2, b2.reshape(1, 64))


# speedup vs baseline: 71.4235x; 1.1272x over previous
import functools

import jax
import jax.numpy as jnp
from jax import lax
from jax.experimental import pallas as pl
from jax.experimental.pallas import tpu as pltpu
from jax.experimental.pallas import tpu_sc as plsc

N = 10000
D_IN = 256
OUT = 64
E = 160000

NC = 2
NS = 16
CH = 128
CHUNKS = 40
CH0 = 54
CH1 = 2 * CHUNKS - CH0
EP = NC * NS * CHUNKS * CH
NP = 10112
STRIPE = NP // NS

BLK = 1000
GRID = N // BLK

_f32 = jnp.float32

_SC_PARAMS = pltpu.CompilerParams(
    needs_layout_passes=False,
    use_tc_tiling_on_sc=False,
)


def _sel_mat(rows, cols):
    if rows < cols:
        return (lax.broadcasted_iota(jnp.int32, (rows, cols), 0)
                == lax.broadcasted_iota(jnp.int32, (rows, cols), 1)
                // (cols // rows)).astype(_f32)
    else:
        return (lax.broadcasted_iota(jnp.int32, (rows, cols), 0)
                // (rows // cols)
                == lax.broadcasted_iota(jnp.int32, (rows, cols), 1)
                ).astype(_f32)


def _tc1_body(x_ref, w1_ref, ats_ref, atd_ref, h_ref, as_ref, ad_ref,
              eas_ref):
    h = jnp.dot(x_ref[...], w1_ref[...], preferred_element_type=_f32)
    h_ref[...] = h
    sel = _sel_mat(64, 8)
    a_s = jnp.dot(h * ats_ref[...], sel, preferred_element_type=_f32)
    a_d = jnp.dot(h * atd_ref[...], sel, preferred_element_type=_f32)
    as_ref[...] = a_s
    ad_ref[...] = a_d
    al = a_s + a_d
    eas_ref[...] = jnp.exp(jnp.where(al >= 0, al, 0.2 * al))


def _tc1(xp, w1, ats, atd):
    blk = NP // 16
    return pl.pallas_call(
        _tc1_body,
        grid=(16,),
        in_specs=[
            pl.BlockSpec((blk, D_IN), lambda i: (i, 0)),
            pl.BlockSpec((D_IN, 64), lambda i: (0, 0)),
            pl.BlockSpec((1, 64), lambda i: (0, 0)),
            pl.BlockSpec((1, 64), lambda i: (0, 0)),
        ],
        out_specs=[
            pl.BlockSpec((blk, 64), lambda i: (i, 0)),
            pl.BlockSpec((blk, 8), lambda i: (i, 0)),
            pl.BlockSpec((blk, 8), lambda i: (i, 0)),
            pl.BlockSpec((blk, 8), lambda i: (i, 0)),
        ],
        out_shape=[
            jax.ShapeDtypeStruct((NP, 64), _f32),
            jax.ShapeDtypeStruct((NP, 8), _f32),
            jax.ShapeDtypeStruct((NP, 8), _f32),
            jax.ShapeDtypeStruct((NP, 8), _f32),
        ],
    )(xp, w1, ats, atd)


def _tc2_body(o10_ref, o11_ref, eas_ref, h1_ref, w2_ref,
              at2s_ref, at2d_ref, b1_ref, h2_ref, as2_ref, ad2_ref):
    sel = _sel_mat(8, 64)
    eas = eas_ref[...]
    o10 = o10_ref[0]
    o11 = o11_ref[0]
    s64 = jnp.dot(o10[:, 64:72] + o11[:, 64:72] + eas, sel,
                  preferred_element_type=_f32)
    num = (o10[:, 0:64] + o11[:, 0:64]
           + h1_ref[...] * jnp.dot(eas, sel, preferred_element_type=_f32))
    o1 = num / (s64 + 1e-16) + b1_ref[...]
    h1p = jnp.where(o1 > 0, o1, jnp.exp(jnp.minimum(o1, 0.0)) - 1.0)
    h2 = jnp.dot(h1p, w2_ref[...], preferred_element_type=_f32)
    h2_ref[...] = h2
    as2_ref[...] = jnp.broadcast_to(
        jnp.sum(h2 * at2s_ref[...], axis=1, keepdims=True), h2.shape[:1] + (8,))
    ad2_ref[...] = jnp.broadcast_to(
        jnp.sum(h2 * at2d_ref[...], axis=1, keepdims=True), h2.shape[:1] + (8,))


def _tc2(o1, eas, h1, w2, at2s, at2d, b1):
    blk = NP // 16
    return pl.pallas_call(
        _tc2_body,
        grid=(16,),
        in_specs=[
            pl.BlockSpec((1, blk, 72), lambda i: (0, i, 0)),
            pl.BlockSpec((1, blk, 72), lambda i: (1, i, 0)),
            pl.BlockSpec((blk, 8), lambda i: (i, 0)),
            pl.BlockSpec((blk, 64), lambda i: (i, 0)),
            pl.BlockSpec((64, 64), lambda i: (0, 0)),
            pl.BlockSpec((1, 64), lambda i: (0, 0)),
            pl.BlockSpec((1, 64), lambda i: (0, 0)),
            pl.BlockSpec((1, 64), lambda i: (0, 0)),
        ],
        out_specs=[
            pl.BlockSpec((blk, 64), lambda i: (i, 0)),
            pl.BlockSpec((blk, 8), lambda i: (i, 0)),
            pl.BlockSpec((blk, 8), lambda i: (i, 0)),
        ],
        out_shape=[
            jax.ShapeDtypeStruct((NP, 64), _f32),
            jax.ShapeDtypeStruct((NP, 8), _f32),
            jax.ShapeDtypeStruct((NP, 8), _f32),
        ],
    )(o1, o1, eas, h1, w2, at2s, at2d, b1)


def _tc3_body(o20_ref, o21_ref, as2_ref, ad2_ref, h2_ref,
              b2_ref, out_ref):
    al = as2_ref[...][:, 0:1] + ad2_ref[...][:, 0:1]
    eas2 = jnp.exp(jnp.where(al >= 0, al, 0.2 * al))
    o20 = o20_ref[0]
    o21 = o21_ref[0]
    s = o20[:, 64:65] + o21[:, 64:65] + eas2
    num = o20[:, 0:64] + o21[:, 0:64] + h2_ref[...] * eas2
    o = num / (s + 1e-16) + b2_ref[...]
    z = o - jnp.max(o, axis=1, keepdims=True)
    out_ref[...] = z - jnp.log(jnp.sum(jnp.exp(z), axis=1, keepdims=True))


def _tc3(o2, as2, ad2, h2, b2):
    return pl.pallas_call(
        _tc3_body,
        grid=(GRID,),
        in_specs=[
            pl.BlockSpec((1, BLK, 72), lambda i: (0, i, 0)),
            pl.BlockSpec((1, BLK, 72), lambda i: (1, i, 0)),
            pl.BlockSpec((BLK, 8), lambda i: (i, 0)),
            pl.BlockSpec((BLK, 8), lambda i: (i, 0)),
            pl.BlockSpec((BLK, 64), lambda i: (i, 0)),
            pl.BlockSpec((1, 64), lambda i: (0, 0)),
        ],
        out_specs=pl.BlockSpec((BLK, 64), lambda i: (i, 0)),
        out_shape=jax.ShapeDtypeStruct((N, 64), _f32),
    )(o2, o2, as2, ad2, h2, b2)


def _sc_edge_pass(nea):
    w = 72

    @functools.partial(
        pl.kernel,
        out_type=jax.ShapeDtypeStruct((NC, NP, w), _f32),
        mesh=plsc.VectorSubcoreMesh(core_axis_name="c", subcore_axis_name="s"),
        compiler_params=_SC_PARAMS,
        scratch_types=[
            pltpu.VMEM_SHARED((NP, w), _f32),
            pltpu.VMEM((2, CH), jnp.int32),
            pltpu.VMEM((2, CH), jnp.int32),
            pltpu.VMEM((2, CH, 8), _f32),
            pltpu.VMEM((2, CH, 8), _f32),
            pltpu.VMEM((2, CH, 64), _f32),
            pltpu.VMEM((2, CH, w), _f32),
            pltpu.VMEM((2, CH), jnp.int32),
            pltpu.SemaphoreType.DMA,
            pltpu.SemaphoreType.DMA,
            pltpu.SemaphoreType.DMA,
            pltpu.SemaphoreType.DMA,
            pltpu.SemaphoreType.DMA,
            pltpu.SemaphoreType.DMA,
            pltpu.SemaphoreType.DMA,
            pltpu.SemaphoreType.DMA,
        ],
    )
    def edge_pass(h_hbm, as_hbm, ad_hbm, src_hbm, dst_hbm, z_hbm, out_hbm,
                  acc_sh, src_v, dst_v, asr_v, adr_v, hr_v, msg_v, dsc_v,
                  *sems):
        cid = lax.axis_index("c")
        sid = lax.axis_index("s")
        r0 = sid * STRIPE
        pltpu.sync_copy(z_hbm, acc_sh.at[pl.ds(r0, STRIPE)])
        plsc.subcore_barrier()

        i16 = lax.broadcasted_iota(jnp.int32, (16,), 0)
        p8 = i16 // 8
        e8 = i16 - 8 * p8
        nch = jnp.where(cid == 0, CH0, CH1)
        base0 = jnp.where(cid == 0, sid * (CH0 * CH),
                          NS * CH0 * CH + sid * (CH1 * CH))

        if nea != 8:
            for b0 in range(2):
                def z_body(e):
                    plsc.store_scatter(msg_v.at[b0], [i16 * 0 + e, 56 + i16],
                                       jnp.zeros((16,), _f32))
                plsc.parallel_loop(0, CH, 1, unroll=4)(z_body)

        def prefetch(c, b):
            base = base0 + c * CH
            pltpu.sync_copy(src_hbm.at[pl.ds(base, CH)], src_v.at[b])
            pltpu.sync_copy(dst_hbm.at[pl.ds(base, CH)], dst_v.at[b])
            pltpu.async_copy(as_hbm.at[src_v.at[b]], asr_v.at[b], sems[3 * b])
            pltpu.async_copy(ad_hbm.at[dst_v.at[b]], adr_v.at[b],
                             sems[3 * b + 1])
            pltpu.async_copy(h_hbm.at[src_v.at[b]], hr_v.at[b],
                             sems[3 * b + 2])

        def compute(b, drain_pred):
            asr_b, adr_b, hr_b = asr_v.at[b], adr_v.at[b], hr_v.at[b]
            msg_b = msg_v.at[b]
            pltpu.make_async_copy(as_hbm.at[src_v.at[b]], asr_b,
                                  sems[3 * b]).wait()
            pltpu.make_async_copy(ad_hbm.at[dst_v.at[b]], adr_b,
                                  sems[3 * b + 1]).wait()

            @pl.when(drain_pred)
            def _():
                pltpu.make_async_copy(msg_b, acc_sh.at[dsc_v.at[b]],
                                      sems[6 + b]).wait()

            if nea == 8:
                def ea_body(v):
                    rows = 2 * v + p8
                    a = (plsc.load_gather(asr_b, [rows, e8])
                         + plsc.load_gather(adr_b, [rows, e8]))
                    a = jnp.where(a >= 0, a, 0.2 * a)
                    plsc.store_scatter(msg_b, [rows, 64 + e8], jnp.exp(a))
                plsc.parallel_loop(0, CH // 2, 1, unroll=4)(ea_body)
            else:
                z16 = i16 * 0
                def ea_body(v):
                    rows = 16 * v + i16
                    a = (plsc.load_gather(asr_b, [rows, z16])
                         + plsc.load_gather(adr_b, [rows, z16]))
                    a = jnp.where(a >= 0, a, 0.2 * a)
                    plsc.store_scatter(msg_b, [rows, z16 + 64], jnp.exp(a))
                plsc.parallel_loop(0, CH // 16, 1, unroll=2)(ea_body)

            pltpu.make_async_copy(h_hbm.at[src_v.at[b]], hr_b,
                                  sems[3 * b + 2]).wait()

            if nea == 8:
                def mul_body(e):
                    erow = i16 * 0 + e
                    for k in range(4):
                        mult = plsc.load_gather(msg_b, [erow, 64 + 2 * k + p8])
                        hv = plsc.load_gather(hr_b, [erow, 16 * k + i16])
                        plsc.store_scatter(msg_b, [erow, 16 * k + i16],
                                           hv * mult)
                plsc.parallel_loop(0, CH, 1, unroll=4)(mul_body)
            else:
                z16 = i16 * 0
                def mul_body(e):
                    erow = i16 * 0 + e
                    mult = plsc.load_gather(msg_b, [erow, z16 + 64])
                    for k in range(4):
                        hv = plsc.load_gather(hr_b, [erow, 16 * k + i16])
                        plsc.store_scatter(msg_b, [erow, 16 * k + i16],
                                           hv * mult)
                plsc.parallel_loop(0, CH, 1, unroll=4)(mul_body)

            for v in range(CH // 16):
                dsc_v[b, pl.ds(16 * v, 16)] = dst_v[b, pl.ds(16 * v, 16)]
            pltpu.async_copy(msg_b, acc_sh.at[dsc_v.at[b]], sems[6 + b],
                             add=True)

        prefetch(0, 0)

        def chunk_body(i, carry):
            j = 2 * i
            prefetch(j + 1, 1)
            compute(0, i > 0)

            @pl.when(j + 2 < nch)
            def _():
                prefetch(j + 2, 0)
            compute(1, i > 0)
            return carry

        lax.fori_loop(0, nch // 2, chunk_body, 0)
        pltpu.make_async_copy(msg_v.at[0], acc_sh.at[dsc_v.at[0]],
                              sems[6]).wait()
        pltpu.make_async_copy(msg_v.at[1], acc_sh.at[dsc_v.at[1]],
                              sems[7]).wait()
        plsc.subcore_barrier()
        pltpu.sync_copy(acc_sh.at[pl.ds(r0, STRIPE)],
                        out_hbm.at[cid, pl.ds(r0, STRIPE)])

    return edge_pass


_sc_layer1 = _sc_edge_pass(8)
_sc_layer2 = _sc_edge_pass(1)


def kernel(x, edge_index, W1, att_src1, att_dst1, b1, W2, att_src2,
           att_dst2, b2):
    src = edge_index[0]
    dst = edge_index[1]
    pad = jnp.full((EP - E,), N, jnp.int32)
    srcp = jnp.concatenate([src, pad])
    dstp = jnp.concatenate([dst, pad])
    z72 = jnp.zeros((STRIPE, 72), _f32)

    xp = jnp.pad(x, ((0, NP - N), (0, 0)))
    h1, as1, ad1, eas1 = _tc1(xp, W1, att_src1.reshape(1, 64),
                              att_dst1.reshape(1, 64))

    o1 = _sc_layer1(h1, as1, ad1, srcp, dstp, z72)

    h2, as2, ad2 = _tc2(o1, eas1, h1, W2, att_src2.reshape(1, 64),
                        att_dst2.reshape(1, 64), b1.reshape(1, 64))

    o2 = _sc_layer2(h2, as2, ad2, srcp, dstp, z72)

    return _tc3(o2, as2, ad2, h2, b2.reshape(1, 64))
